# Initial kernel scaffold; baseline (speedup 1.0000x reference)
import jax
import jax.numpy as jnp
from jax.experimental import pallas as pl


def kernel(X0, X1, X2, L0_idx, L0_val, L1u_idx, L1u_val, L1d_idx, L1d_val, L2_idx, L2_val, batch0, batch1, batch2, params):
    raise NotImplementedError("write your pallas kernel here")


# speedup vs baseline: 10.4482x; 10.4482x over previous
import functools

import jax
import jax.numpy as jnp
from jax import lax
from jax.experimental import pallas as pl
from jax.experimental.pallas import tpu as pltpu
from jax.experimental.pallas import tpu_sc as plsc

F = 30
FP = 32
OUTD = 10
NC = 2
NS = 16
NW = NC * NS
K = 4
EBLK = NW * K * 128

_mesh = plsc.VectorSubcoreMesh(
    core_axis_name="c", subcore_axis_name="s", num_cores=NC, num_subcores=NS)


def _e_pad(e):
    return ((e + EBLK - 1) // EBLK) * EBLK


def _n_pad(n):
    return ((n + 255) // 256) * 256


def _worker_id():
    return lax.axis_index("c") * NS + lax.axis_index("s")


def _zero_vmem(v):
    def zb(i, carry):
        v[pl.ds(i * 16, 16)] = jnp.zeros((16,), jnp.float32)
        return carry

    lax.fori_loop(0, v.shape[0] // 16, zb, 0)


def _reduce_tiles(acc_ref, tmp_v, dall_sh, out_hbm, c, s, zlen):
    pltpu.sync_copy(acc_ref, dall_sh.at[s])
    plsc.subcore_barrier()
    pltpu.sync_copy(dall_sh.at[0, pl.ds(s * zlen, zlen)],
                    acc_ref.at[pl.ds(0, zlen)])
    for t in range(1, NS):
        pltpu.sync_copy(dall_sh.at[t, pl.ds(s * zlen, zlen)], tmp_v)

        def ab(i, carry):
            sl = pl.ds(i * 16, 16)
            acc_ref[sl] = acc_ref[sl] + tmp_v[sl]
            return carry

        lax.fori_loop(0, zlen // 16, ab, 0)
    pltpu.sync_copy(acc_ref.at[pl.ds(0, zlen)],
                    out_hbm.at[c, pl.ds(s * zlen, zlen)])


def _zero_shared(z_v, sh, rows_per_tile):
    s = lax.axis_index("s")
    if len(z_v.shape) == 1:
        for i in range(z_v.shape[0] // 16):
            z_v[pl.ds(i * 16, 16)] = jnp.zeros((16,), jnp.float32)
    else:
        for r in range(z_v.shape[0]):
            for i in range(z_v.shape[1] // 16):
                z_v[r, pl.ds(i * 16, 16)] = jnp.zeros((16,), jnp.float32)

    nz = rows_per_tile // z_v.shape[0]

    def zb(i, carry):
        pltpu.sync_copy(z_v, sh.at[pl.ds(s * rows_per_tile + i * z_v.shape[0],
                                         z_v.shape[0])])
        return carry

    lax.fori_loop(0, nz, zb, 0)


@functools.lru_cache(maxsize=None)
def _build_dseg(e_pad, n_pad):
    rows = e_pad // 128
    rows_w = rows // NW
    groups = rows_w // K
    zlen = n_pad // NS

    @functools.partial(
        pl.kernel,
        out_type=jax.ShapeDtypeStruct((NC, n_pad), jnp.float32),
        mesh=_mesh,
        compiler_params=pltpu.CompilerParams(
            needs_layout_passes=False, use_tc_tiling_on_sc=False),
        scratch_types=[
            pltpu.VMEM((K, 128), jnp.int32),
            pltpu.VMEM((K, 128), jnp.float32),
            pltpu.VMEM((n_pad,), jnp.float32),
            pltpu.VMEM((zlen,), jnp.float32),
            pltpu.VMEM_SHARED((NS, n_pad), jnp.float32),
        ],
    )
    def dker(row_hbm, val_hbm, dpart_hbm, idx_v, w_v, d_v, tmp_v, dall_sh):
        c = lax.axis_index("c")
        s = lax.axis_index("s")
        wid = _worker_id()
        _zero_vmem(d_v)
        base = wid * rows_w

        def gbody(g, carry):
            r0 = base + g * K
            pltpu.sync_copy(row_hbm.at[pl.ds(r0, K)], idx_v)
            pltpu.sync_copy(val_hbm.at[pl.ds(r0, K)], w_v)
            for j in range(K):
                for i in range(8):
                    sl = (j, pl.ds(i * 16, 16))
                    plsc.addupdate_scatter(d_v, [idx_v[sl]], jnp.abs(w_v[sl]))
            return carry

        lax.fori_loop(0, groups, gbody, 0)
        _reduce_tiles(d_v, tmp_v, dall_sh, dpart_hbm, c, s, zlen)

    return dker


@functools.lru_cache(maxsize=None)
def _build_pvals(e_pad, n):
    rows = e_pad // 128
    rows_w = rows // NW
    groups = rows_w // K

    @functools.partial(
        pl.kernel,
        out_type=jax.ShapeDtypeStruct((rows, 128), jnp.float32),
        mesh=_mesh,
        compiler_params=pltpu.CompilerParams(
            needs_layout_passes=False, use_tc_tiling_on_sc=False),
        scratch_types=[
            pltpu.VMEM((n,), jnp.float32),
            pltpu.VMEM((K, 128), jnp.int32),
            pltpu.VMEM((K, 128), jnp.int32),
            pltpu.VMEM((K, 128), jnp.float32),
        ],
    )
    def pker(row_hbm, col_hbm, val_hbm, dinv_hbm, w_hbm, dinv_v, idxr, idxc, v_v):
        wid = _worker_id()
        pltpu.sync_copy(dinv_hbm, dinv_v)
        base = wid * rows_w

        def gbody(g, carry):
            r0 = base + g * K
            pltpu.sync_copy(row_hbm.at[pl.ds(r0, K)], idxr)
            pltpu.sync_copy(col_hbm.at[pl.ds(r0, K)], idxc)
            pltpu.sync_copy(val_hbm.at[pl.ds(r0, K)], v_v)
            for j in range(K):
                for i in range(8):
                    sl = (j, pl.ds(i * 16, 16))
                    rv = plsc.load_gather(dinv_v, [idxr[sl]])
                    cv = plsc.load_gather(dinv_v, [idxc[sl]])
                    v_v[sl] = v_v[sl] * rv * cv
            pltpu.sync_copy(v_v, w_hbm.at[pl.ds(r0, K)])
            return carry

        lax.fori_loop(0, groups, gbody, 0)

    return pker


@functools.lru_cache(maxsize=None)
def _build_softmax(e_pad, e_real, n, n_pad):
    rows = e_pad // 128
    rows_w = rows // NW
    groups = rows_w // K
    zlen = n_pad // NS

    @functools.partial(
        pl.kernel,
        out_type=(jax.ShapeDtypeStruct((rows, 128), jnp.float32),
                  jax.ShapeDtypeStruct((NC, n_pad), jnp.float32)),
        mesh=_mesh,
        compiler_params=pltpu.CompilerParams(
            needs_layout_passes=False, use_tc_tiling_on_sc=False),
        scratch_types=[
            pltpu.VMEM((n,), jnp.float32),
            pltpu.VMEM((n,), jnp.float32),
            pltpu.VMEM((n,), jnp.float32),
            pltpu.VMEM((K, 128), jnp.int32),
            pltpu.VMEM((K, 128), jnp.int32),
            pltpu.VMEM((K, 128), jnp.float32),
            pltpu.VMEM((n_pad,), jnp.float32),
            pltpu.VMEM((zlen,), jnp.float32),
            pltpu.VMEM_SHARED((NS, n_pad), jnp.float32),
        ],
    )
    def smker(src_hbm, dst_hbm, as_hbm, ad_hbm, c_hbm, ex_hbm, den_hbm,
              as_v, ad_v, c_v, idxs, idxd, ex_v, den_v, tmp_v, dall_sh):
        c = lax.axis_index("c")
        s = lax.axis_index("s")
        wid = _worker_id()
        pltpu.sync_copy(as_hbm, as_v)
        pltpu.sync_copy(ad_hbm, ad_v)
        pltpu.sync_copy(c_hbm, c_v)
        _zero_vmem(den_v)
        base = wid * rows_w
        iota = jnp.arange(16, dtype=jnp.int32)

        def gbody(g, carry):
            r0 = base + g * K
            pltpu.sync_copy(src_hbm.at[pl.ds(r0, K)], idxs)
            pltpu.sync_copy(dst_hbm.at[pl.ds(r0, K)], idxd)
            for j in range(K):
                for i in range(8):
                    sl = (j, pl.ds(i * 16, 16))
                    di = idxd[sl]
                    sv = plsc.load_gather(as_v, [idxs[sl]])
                    dv = plsc.load_gather(ad_v, [di])
                    cv = plsc.load_gather(c_v, [di])
                    t = sv + dv
                    e = jnp.maximum(t, 0.2 * t)
                    ex = jnp.exp(e - cv)
                    eid = (r0 + j) * 128 + (i * 16) + iota
                    ex = jnp.where(eid < e_real, ex, 0.0)
                    ex_v[sl] = ex
                    plsc.addupdate_scatter(den_v, [di], ex)
            pltpu.sync_copy(ex_v, ex_hbm.at[pl.ds(r0, K)])
            return carry

        lax.fori_loop(0, groups, gbody, 0)
        _reduce_tiles(den_v, tmp_v, dall_sh, den_hbm, c, s, zlen)

    return smker


@functools.lru_cache(maxsize=None)
def _build_rows(e_pad, n, n_pad, gat):
    rows = e_pad // 128
    rows_w = rows // NW
    groups = rows_w // K
    zlen = n_pad // NS
    EB = K * 128

    scratch = [
        pltpu.VMEM((K, 128), jnp.int32),
        pltpu.VMEM((K, 128), jnp.int32),
        pltpu.VMEM((K, 128), jnp.float32),
        pltpu.VMEM((EB, FP), jnp.float32),
        pltpu.VMEM((8, FP), jnp.float32),
        pltpu.VMEM_SHARED((n_pad, FP), jnp.float32),
        pltpu.SemaphoreType.DMA,
    ]
    if gat:
        scratch += [pltpu.VMEM((n_pad,), jnp.float32),
                    pltpu.VMEM((n_pad,), jnp.float32)]

    @functools.partial(
        pl.kernel,
        out_type=jax.ShapeDtypeStruct((NC, n_pad, FP), jnp.float32),
        mesh=_mesh,
        compiler_params=pltpu.CompilerParams(
            needs_layout_passes=False, use_tc_tiling_on_sc=False),
        scratch_types=scratch,
    )
    def rker(*refs):
        if gat:
            (t_hbm, src_hbm, dst_hbm, w_hbm, den_hbm, out_hbm,
             idxs, idxd, w_v, rows_v, z_v, out_sh, sem, den0_v, den1_v) = refs
        else:
            (t_hbm, src_hbm, dst_hbm, w_hbm, out_hbm,
             idxs, idxd, w_v, rows_v, z_v, out_sh, sem) = refs
        c = lax.axis_index("c")
        s = lax.axis_index("s")
        wid = _worker_id()
        _zero_shared(z_v, out_sh, zlen)
        if gat:
            pltpu.sync_copy(den_hbm.at[0], den0_v)
            pltpu.sync_copy(den_hbm.at[1], den1_v)

            def dbody(i, carry):
                sl = pl.ds(i * 16, 16)
                den0_v[sl] = 1.0 / (den0_v[sl] + den1_v[sl] + 1e-16)
                return carry

            lax.fori_loop(0, n_pad // 16, dbody, 0)
        plsc.subcore_barrier()
        base = wid * rows_w
        iota = jnp.arange(16, dtype=jnp.int32)

        def gbody(g, carry):
            r0 = base + g * K
            pltpu.sync_copy(src_hbm.at[pl.ds(r0, K)], idxs)
            pltpu.sync_copy(dst_hbm.at[pl.ds(r0, K)], idxd)
            pltpu.sync_copy(w_hbm.at[pl.ds(r0, K)], w_v)
            if gat:
                for j in range(K):
                    for i in range(8):
                        sl = (j, pl.ds(i * 16, 16))
                        dv = plsc.load_gather(den0_v, [idxd[sl]])
                        w_v[sl] = w_v[sl] * dv
            cps = [pltpu.async_copy(t_hbm.at[idxs.at[j]],
                                    rows_v.at[pl.ds(j * 128, 128)], sem)
                   for j in range(K)]
            for cp in cps:
                cp.wait()

            def sbody(q, carry2):
                ridx = q * 16 + iota
                jq = ridx // 128
                iq = lax.rem(ridx, 128)
                wv = plsc.load_gather(w_v, [jq, iq])
                for col in range(FP):
                    cc = jnp.full((16,), col, jnp.int32)
                    rv = plsc.load_gather(rows_v, [ridx, cc])
                    plsc.store_scatter(rows_v, [ridx, cc], rv * wv)
                return carry2

            lax.fori_loop(0, EB // 16, sbody, 0)
            for j in range(K):
                pltpu.sync_copy(rows_v.at[pl.ds(j * 128, 128)],
                                out_sh.at[idxd.at[j]], add=True)
            return carry

        lax.fori_loop(0, groups, gbody, 0)
        plsc.subcore_barrier()
        pltpu.sync_copy(out_sh.at[pl.ds(s * zlen, zlen)],
                        out_hbm.at[c, pl.ds(s * zlen, zlen)])

    return rker


_BS = 1000


def _full(shp):
    return pl.BlockSpec(shp, lambda i: (0,) * len(shp))


@functools.lru_cache(maxsize=None)
def _build_affine_first(n, din):
    nb = n // _BS

    def body(x_ref, W_ref, b_ref, pW_ref, pb_ref, h_o, hp_o):
        x = x_ref[...]
        h_o[...] = jnp.dot(x, W_ref[...],
                           preferred_element_type=jnp.float32) + b_ref[...]
        hp_o[...] = jnp.dot(x, pW_ref[...],
                            preferred_element_type=jnp.float32) + pb_ref[...]

    in_specs = [pl.BlockSpec((_BS, din), lambda i: (i, 0)),
                _full((din, FP)), _full((1, FP)), _full((din, FP)),
                _full((1, FP))]
    out_specs = [pl.BlockSpec((_BS, FP), lambda i: (i, 0)),
                 pl.BlockSpec((_BS, FP), lambda i: (i, 0))]
    out = [jax.ShapeDtypeStruct((n, FP), jnp.float32),
           jax.ShapeDtypeStruct((n, FP), jnp.float32)]
    return pl.pallas_call(body, grid=(nb,), in_specs=in_specs,
                          out_specs=out_specs, out_shape=out)


@functools.lru_cache(maxsize=None)
def _build_affine_next(n, n_pad, nparts):
    nb = n // _BS

    def body(*refs):
        parts = refs[:nparts]
        W_ref, b_ref, pW_ref, pb_ref = refs[nparts:nparts + 4]
        x_o, h_o, hp_o = refs[nparts + 4:]
        acc = parts[0][0] + parts[0][1]
        for p in parts[1:]:
            acc = acc + p[0] + p[1]
        x = jnp.maximum(acc, 0.0)
        x_o[...] = x
        h_o[...] = jnp.dot(x, W_ref[...],
                           preferred_element_type=jnp.float32) + b_ref[...]
        hp_o[...] = jnp.dot(x, pW_ref[...],
                            preferred_element_type=jnp.float32) + pb_ref[...]

    part_spec = pl.BlockSpec((NC, _BS, FP), lambda i: (0, i, 0))
    in_specs = [part_spec] * nparts + [
        _full((FP, FP)), _full((1, FP)), _full((FP, FP)), _full((1, FP))]
    out_specs = [pl.BlockSpec((_BS, FP), lambda i: (i, 0))] * 3
    out = [jax.ShapeDtypeStruct((n, FP), jnp.float32),
           jax.ShapeDtypeStruct((n, FP), jnp.float32),
           jax.ShapeDtypeStruct((n, FP), jnp.float32)]
    return pl.pallas_call(body, grid=(nb,), in_specs=in_specs,
                          out_specs=out_specs, out_shape=out)


@functools.lru_cache(maxsize=None)
def _build_attn(n):
    def body(h_ref, asrc_ref, adst_ref, as_o, ad_o, c_o):
        h = h_ref[...]
        a_s = (h @ asrc_ref[0])[None, :]
        a_d = (h @ adst_ref[0])[None, :]
        ms = jnp.max(a_s)
        t = ms + a_d
        as_o[...] = a_s
        ad_o[...] = a_d
        c_o[...] = jnp.maximum(t, 0.2 * t)

    out = [jax.ShapeDtypeStruct((1, n), jnp.float32)] * 3
    return pl.pallas_call(body, out_shape=out)


@functools.lru_cache(maxsize=None)
def _build_dinv(n, n_pad):
    def body(dp_ref, out_ref):
        d = dp_ref[0, :n] + dp_ref[1, :n]
        out_ref[...] = lax.rsqrt(d + 1e-12)[None, :]

    return pl.pallas_call(body, out_shape=jax.ShapeDtypeStruct((1, n), jnp.float32))


@functools.lru_cache(maxsize=None)
def _build_cvec(n):
    def body(as_ref, ad_ref, c_o):
        ms = jnp.max(as_ref[...])
        t = ms + ad_ref[...]
        c_o[...] = jnp.maximum(t, 0.2 * t)

    return pl.pallas_call(body, out_shape=jax.ShapeDtypeStruct((1, n), jnp.float32))


@functools.lru_cache(maxsize=None)
def _build_head(n, n_pad, nparts):
    nb = n // _BS

    def body(*refs):
        x1_ref, x2_ref = refs[:2]
        parts = refs[2:2 + nparts]
        Wa, Wb, Wc, b4 = refs[2 + nparts:2 + nparts + 4]
        out_ref = refs[-1]
        i = pl.program_id(0)
        acc = parts[0][0] + parts[0][1]
        for p in parts[1:]:
            acc = acc + p[0] + p[1]
        x3 = jnp.maximum(acc, 0.0)
        x4 = (jnp.dot(x1_ref[...], Wa[...], preferred_element_type=jnp.float32)
              + jnp.dot(x2_ref[...], Wb[...], preferred_element_type=jnp.float32)
              + jnp.dot(x3, Wc[...], preferred_element_type=jnp.float32))

        @pl.when(i == 0)
        def _():
            out_ref[...] = jnp.zeros_like(out_ref)

        out_ref[...] += jnp.sum(x4, axis=0, keepdims=True)

        @pl.when(i == nb - 1)
        def _():
            out_ref[...] = out_ref[...] * (1.0 / n) + b4[...]

    part_spec = pl.BlockSpec((NC, _BS, FP), lambda i: (0, i, 0))
    in_specs = [pl.BlockSpec((_BS, FP), lambda i: (i, 0))] * 2 + \
        [part_spec] * nparts + [
        _full((FP, OUTD)), _full((FP, OUTD)), _full((FP, OUTD)),
        _full((1, OUTD))]
    return pl.pallas_call(
        body, grid=(nb,), in_specs=in_specs,
        out_specs=pl.BlockSpec((1, OUTD), lambda i: (0, 0)),
        out_shape=jax.ShapeDtypeStruct((1, OUTD), jnp.float32))


def _final_combine(y0, y1, y2, Wc, bc):
    def body(y0_ref, y1_ref, y2_ref, w0, w1, w2, b_ref, out_ref):
        y = (jnp.dot(y0_ref[...], w0[...], preferred_element_type=jnp.float32)
             + jnp.dot(y1_ref[...], w1[...], preferred_element_type=jnp.float32)
             + jnp.dot(y2_ref[...], w2[...], preferred_element_type=jnp.float32)
             + b_ref[...])
        z = y - jnp.max(y)
        e = jnp.exp(z)
        out_ref[...] = e / jnp.sum(e)

    return pl.pallas_call(
        body, out_shape=jax.ShapeDtypeStruct((1, OUTD), jnp.float32))(
        y0, y1, y2, Wc[0:10], Wc[10:20], Wc[20:30], bc[None, :])


def _pad_edge_arr(a, e_pad, dtype):
    E = a.shape[0]
    a = jnp.pad(a.astype(dtype), (0, e_pad - E))
    return a.reshape(e_pad // 128, 128)


def _pad_w(w, fout=FP):
    return jnp.pad(w, ((0, 0), (0, fout - w.shape[1])))


def _pad_w2(w):
    return jnp.pad(w, ((0, FP - w.shape[0]), (0, FP - w.shape[1])))


def _pad_v(v):
    return jnp.pad(v, (0, FP - v.shape[0]))


def _san_weights(p, first):
    ld = p["l_d"]
    W = _pad_w(ld["W"]) if first else _pad_w2(ld["W"])
    pW = _pad_w(p["p_W"]) if first else _pad_w2(p["p_W"])
    return (W, _pad_v(ld["b"])[None, :], pW, _pad_v(p["p_b"])[None, :],
            _pad_v(ld["a_src"])[None, :], _pad_v(ld["a_dst"])[None, :])


def _run_level(X, params, lvl, gat_idx_list, p_idx, p_val, n):
    n_pad = _n_pad(n)
    ep_p = _e_pad(p_idx.shape[1])
    prow = _pad_edge_arr(p_idx[0], ep_p, jnp.int32)
    pcol = _pad_edge_arr(p_idx[1], ep_p, jnp.int32)
    pval = _pad_edge_arr(p_val, ep_p, jnp.float32)
    gats = []
    for gi in gat_idx_list:
        ep = _e_pad(gi.shape[1])
        gats.append((_pad_edge_arr(gi[0], ep, jnp.int32),
                     _pad_edge_arr(gi[1], ep, jnp.int32),
                     ep, gi.shape[1]))

    dpart = _build_dseg(ep_p, n_pad)(prow, pval)

    xs = []
    pw = None
    h = hp = a_s = a_d = cvec = None
    for k in (1, 2, 3):
        p = params["l%d_%d" % (lvl, k)]
        if k == 1:
            W, b, pW, pb, asrc, adst = _san_weights(p, True)
            h, hp = _build_affine_first(n, X.shape[1])(X, W, b, pW, pb)
            a_s, a_d, cvec = _build_attn(n)(h, asrc, adst)
            dinv = _build_dinv(n, n_pad)(dpart)
            pw = _build_pvals(ep_p, n)(prow, pcol, pval, dinv.reshape(n))
        parts = [_build_rows(ep_p, n, n_pad, False)(hp, pcol, prow, pw)]
        for (gsrc, gdst, ep, e_real) in gats:
            ex, den = _build_softmax(ep, e_real, n, n_pad)(
                gsrc, gdst, a_s.reshape(n), a_d.reshape(n), cvec.reshape(n))
            parts.append(_build_rows(ep, n, n_pad, True)(h, gsrc, gdst, ex, den))
        if k < 3:
            p2 = params["l%d_%d" % (lvl, k + 1)]
            W, b, pW, pb, asrc, adst = _san_weights(p2, False)
            x, h, hp = _build_affine_next(n, n_pad, len(parts))(
                *parts, W, b, pW, pb)
            a_s, a_d, cvec = _build_attn(n)(h, asrc, adst)
            xs.append(x)
        else:
            lin = params["l%d_4" % lvl]
            W4 = lin["W"]
            Wa = jnp.pad(W4[0:30], ((0, 2), (0, 0)))
            Wb = jnp.pad(W4[30:60], ((0, 2), (0, 0)))
            Wc = jnp.pad(W4[60:90], ((0, 2), (0, 0)))
            return _build_head(n, n_pad, len(parts))(
                xs[0], xs[1], *parts, Wa, Wb, Wc, lin["b"][None, :])


def kernel(X0, X1, X2, L0_idx, L0_val, L1u_idx, L1u_val, L1d_idx, L1d_val,
           L2_idx, L2_val, batch0, batch1, batch2, params):
    L1_idx = jnp.concatenate([L1u_idx, L1d_idx], axis=1)
    L1_val = jnp.concatenate([L1u_val, L1d_val], axis=0)
    y0 = _run_level(X0, params, 0, [L0_idx], L0_idx, L0_val, X0.shape[0])
    y1 = _run_level(X1, params, 1, [L1u_idx, L1d_idx], L1_idx, L1_val, X1.shape[0])
    y2 = _run_level(X2, params, 2, [L2_idx], L2_idx, L2_val, X2.shape[0])
    comb = params["combined"]
    return _final_combine(y0, y1, y2, comb["W"], comb["b"])


# speedup vs baseline: 10.9328x; 1.0464x over previous
import functools

import jax
import jax.numpy as jnp
from jax import lax
from jax.experimental import pallas as pl
from jax.experimental.pallas import tpu as pltpu
from jax.experimental.pallas import tpu_sc as plsc

F = 30
FP = 32
OUTD = 10
NC = 2
NS = 16
NW = NC * NS
K = 4
EBLK = 2 * NW * K * 128

_mesh = plsc.VectorSubcoreMesh(
    core_axis_name="c", subcore_axis_name="s", num_cores=NC, num_subcores=NS)


def _e_pad(e):
    return ((e + EBLK - 1) // EBLK) * EBLK


def _n_pad(n):
    return ((n + 255) // 256) * 256


def _worker_id():
    return lax.axis_index("c") * NS + lax.axis_index("s")


def _zero_vmem(v):
    def zb(i, carry):
        v[pl.ds(i * 16, 16)] = jnp.zeros((16,), jnp.float32)
        return carry

    lax.fori_loop(0, v.shape[0] // 16, zb, 0)


def _reduce_tiles(acc_ref, tmp_v, dall_sh, out_hbm, c, s, zlen):
    pltpu.sync_copy(acc_ref, dall_sh.at[s])
    plsc.subcore_barrier()
    pltpu.sync_copy(dall_sh.at[0, pl.ds(s * zlen, zlen)],
                    acc_ref.at[pl.ds(0, zlen)])
    for t in range(1, NS):
        pltpu.sync_copy(dall_sh.at[t, pl.ds(s * zlen, zlen)], tmp_v)

        def ab(i, carry):
            sl = pl.ds(i * 16, 16)
            acc_ref[sl] = acc_ref[sl] + tmp_v[sl]
            return carry

        lax.fori_loop(0, zlen // 16, ab, 0)
    pltpu.sync_copy(acc_ref.at[pl.ds(0, zlen)],
                    out_hbm.at[c, pl.ds(s * zlen, zlen)])


def _zero_shared(z_v, sh, rows_per_tile):
    s = lax.axis_index("s")
    if len(z_v.shape) == 1:
        for i in range(z_v.shape[0] // 16):
            z_v[pl.ds(i * 16, 16)] = jnp.zeros((16,), jnp.float32)
    else:
        for r in range(z_v.shape[0]):
            for i in range(z_v.shape[1] // 16):
                z_v[r, pl.ds(i * 16, 16)] = jnp.zeros((16,), jnp.float32)

    nz = rows_per_tile // z_v.shape[0]

    def zb(i, carry):
        pltpu.sync_copy(z_v, sh.at[pl.ds(s * rows_per_tile + i * z_v.shape[0],
                                         z_v.shape[0])])
        return carry

    lax.fori_loop(0, nz, zb, 0)


@functools.lru_cache(maxsize=None)
def _build_dseg(e_pad, n_pad):
    rows = e_pad // 128
    rows_w = rows // NW
    groups = rows_w // K
    zlen = n_pad // NS

    @functools.partial(
        pl.kernel,
        out_type=jax.ShapeDtypeStruct((NC, n_pad), jnp.float32),
        mesh=_mesh,
        compiler_params=pltpu.CompilerParams(
            needs_layout_passes=False, use_tc_tiling_on_sc=False),
        scratch_types=[
            pltpu.VMEM((K, 2, 128), jnp.int32),
            pltpu.VMEM((K, 128), jnp.float32),
            pltpu.VMEM((n_pad,), jnp.float32),
            pltpu.VMEM((zlen,), jnp.float32),
            pltpu.VMEM_SHARED((NS, n_pad), jnp.float32),
        ],
    )
    def dker(pack_hbm, val_hbm, dpart_hbm, idx_v, w_v, d_v, tmp_v, dall_sh):
        c = lax.axis_index("c")
        s = lax.axis_index("s")
        wid = _worker_id()
        _zero_vmem(d_v)
        base = wid * rows_w

        def gbody(g, carry):
            r0 = base + g * K
            pltpu.sync_copy(pack_hbm.at[pl.ds(r0, K)], idx_v)
            pltpu.sync_copy(val_hbm.at[pl.ds(r0, K)], w_v)
            for j in range(K):
                for i in range(8):
                    sl16 = pl.ds(i * 16, 16)
                    plsc.addupdate_scatter(
                        d_v, [idx_v[j, 1, sl16]], jnp.abs(w_v[j, sl16]))
            return carry

        lax.fori_loop(0, groups, gbody, 0)
        _reduce_tiles(d_v, tmp_v, dall_sh, dpart_hbm, c, s, zlen)

    return dker


@functools.lru_cache(maxsize=None)
def _build_pvals(e_pad, n):
    rows = e_pad // 128
    rows_w = rows // NW
    groups = rows_w // K

    @functools.partial(
        pl.kernel,
        out_type=jax.ShapeDtypeStruct((rows, 128), jnp.float32),
        mesh=_mesh,
        compiler_params=pltpu.CompilerParams(
            needs_layout_passes=False, use_tc_tiling_on_sc=False),
        scratch_types=[
            pltpu.VMEM((n,), jnp.float32),
            pltpu.VMEM((K, 2, 128), jnp.int32),
            pltpu.VMEM((K, 128), jnp.float32),
        ],
    )
    def pker(pack_hbm, val_hbm, dinv_hbm, w_hbm, dinv_v, idx_v, v_v):
        wid = _worker_id()
        pltpu.sync_copy(dinv_hbm, dinv_v)
        base = wid * rows_w

        def gbody(g, carry):
            r0 = base + g * K
            pltpu.sync_copy(pack_hbm.at[pl.ds(r0, K)], idx_v)
            pltpu.sync_copy(val_hbm.at[pl.ds(r0, K)], v_v)
            for j in range(K):
                for i in range(8):
                    sl16 = pl.ds(i * 16, 16)
                    rv = plsc.load_gather(dinv_v, [idx_v[j, 1, sl16]])
                    cv = plsc.load_gather(dinv_v, [idx_v[j, 0, sl16]])
                    v_v[j, sl16] = v_v[j, sl16] * rv * cv
            pltpu.sync_copy(v_v, w_hbm.at[pl.ds(r0, K)])
            return carry

        lax.fori_loop(0, groups, gbody, 0)

    return pker


@functools.lru_cache(maxsize=None)
def _build_softmax(e_pad, e_real, n, n_pad):
    rows = e_pad // 128
    rows_w = rows // NW
    groups = rows_w // K
    zlen = n_pad // NS

    @functools.partial(
        pl.kernel,
        out_type=(jax.ShapeDtypeStruct((rows, 128), jnp.float32),
                  jax.ShapeDtypeStruct((NC, n_pad), jnp.float32)),
        mesh=_mesh,
        compiler_params=pltpu.CompilerParams(
            needs_layout_passes=False, use_tc_tiling_on_sc=False),
        scratch_types=[
            pltpu.VMEM((n,), jnp.float32),
            pltpu.VMEM((n,), jnp.float32),
            pltpu.VMEM((n,), jnp.float32),
            pltpu.VMEM((K, 2, 128), jnp.int32),
            pltpu.VMEM((K, 128), jnp.float32),
            pltpu.VMEM((n_pad,), jnp.float32),
            pltpu.VMEM((zlen,), jnp.float32),
            pltpu.VMEM_SHARED((NS, n_pad), jnp.float32),
        ],
    )
    def smker(pack_hbm, as_hbm, ad_hbm, c_hbm, ex_hbm, den_hbm,
              as_v, ad_v, c_v, idx_v, ex_v, den_v, tmp_v, dall_sh):
        c = lax.axis_index("c")
        s = lax.axis_index("s")
        wid = _worker_id()
        pltpu.sync_copy(as_hbm, as_v)
        pltpu.sync_copy(ad_hbm, ad_v)
        pltpu.sync_copy(c_hbm, c_v)
        _zero_vmem(den_v)
        base = wid * rows_w
        iota = jnp.arange(16, dtype=jnp.int32)

        def gbody(g, carry):
            r0 = base + g * K
            pltpu.sync_copy(pack_hbm.at[pl.ds(r0, K)], idx_v)
            for j in range(K):
                for i in range(8):
                    sl16 = pl.ds(i * 16, 16)
                    di = idx_v[j, 1, sl16]
                    sv = plsc.load_gather(as_v, [idx_v[j, 0, sl16]])
                    dv = plsc.load_gather(ad_v, [di])
                    cv = plsc.load_gather(c_v, [di])
                    t = sv + dv
                    e = jnp.maximum(t, 0.2 * t)
                    ex = jnp.exp(e - cv)
                    eid = (r0 + j) * 128 + (i * 16) + iota
                    ex = jnp.where(eid < e_real, ex, 0.0)
                    ex_v[j, sl16] = ex
                    plsc.addupdate_scatter(den_v, [di], ex)
            pltpu.sync_copy(ex_v, ex_hbm.at[pl.ds(r0, K)])
            return carry

        lax.fori_loop(0, groups, gbody, 0)
        _reduce_tiles(den_v, tmp_v, dall_sh, den_hbm, c, s, zlen)

    return smker


@functools.lru_cache(maxsize=None)
def _build_rows(e_pad, n, n_pad, gat):
    rows = e_pad // 128
    rows_w = rows // NW
    groups = rows_w // K
    zlen = n_pad // NS
    EB = K * 128

    scratch = [
        pltpu.VMEM((K, 2, 128), jnp.int32),
        pltpu.VMEM((K, 2, 128), jnp.int32),
        pltpu.VMEM((K, 128), jnp.float32),
        pltpu.VMEM((K, 128), jnp.float32),
        pltpu.VMEM((EB, FP), jnp.float32),
        pltpu.VMEM((EB, FP), jnp.float32),
        pltpu.VMEM((8, FP), jnp.float32),
        pltpu.VMEM_SHARED((n_pad, FP), jnp.float32),
        pltpu.SemaphoreType.DMA,
        pltpu.SemaphoreType.DMA,
        pltpu.SemaphoreType.DMA,
        pltpu.SemaphoreType.DMA,
        pltpu.SemaphoreType.DMA,
        pltpu.SemaphoreType.DMA,
    ]
    if gat:
        scratch += [pltpu.VMEM((n_pad,), jnp.float32),
                    pltpu.VMEM((n_pad,), jnp.float32)]

    @functools.partial(
        pl.kernel,
        out_type=jax.ShapeDtypeStruct((NC, n_pad, FP), jnp.float32),
        mesh=_mesh,
        compiler_params=pltpu.CompilerParams(
            needs_layout_passes=False, use_tc_tiling_on_sc=False),
        scratch_types=scratch,
    )
    def rker(*refs):
        if gat:
            (t_hbm, pack_hbm, w_hbm, den_hbm, out_hbm,
             ip0, ip1, w0, w1, r0_, r1_, z_v, out_sh,
             is0, is1, gs0, gs1, ss0, ss1, den0_v, den1_v) = refs
        else:
            (t_hbm, pack_hbm, w_hbm, out_hbm,
             ip0, ip1, w0, w1, r0_, r1_, z_v, out_sh,
             is0, is1, gs0, gs1, ss0, ss1) = refs
        ip = (ip0, ip1)
        wv = (w0, w1)
        rv_ = (r0_, r1_)
        isem = (is0, is1)
        gsem = (gs0, gs1)
        ssem = (ss0, ss1)
        c = lax.axis_index("c")
        s = lax.axis_index("s")
        wid = _worker_id()
        _zero_shared(z_v, out_sh, zlen)
        if gat:
            pltpu.sync_copy(den_hbm.at[0], den0_v)
            pltpu.sync_copy(den_hbm.at[1], den1_v)

            def dbody(i, carry):
                sl = pl.ds(i * 16, 16)
                den0_v[sl] = 1.0 / (den0_v[sl] + den1_v[sl] + 1e-16)
                return carry

            lax.fori_loop(0, n_pad // 16, dbody, 0)
        plsc.subcore_barrier()
        base = wid * rows_w
        iota = jnp.arange(16, dtype=jnp.int32)

        def issue_inputs(b, g):
            r0 = base + g * K
            pltpu.async_copy(pack_hbm.at[pl.ds(r0, K)], ip[b], isem[b])
            pltpu.async_copy(w_hbm.at[pl.ds(r0, K)], wv[b], isem[b])

        def wait_inputs(b):
            pltpu.make_async_copy(pack_hbm.at[pl.ds(0, K)], ip[b],
                                  isem[b]).wait()
            pltpu.make_async_copy(w_hbm.at[pl.ds(0, K)], wv[b],
                                  isem[b]).wait()

        def issue_gathers(b):
            for j in range(K):
                pltpu.async_copy(t_hbm.at[ip[b].at[j, 0]],
                                 rv_[b].at[pl.ds(j * 128, 128)], gsem[b])

        def wait_gathers(b):
            for j in range(K):
                pltpu.make_async_copy(t_hbm.at[ip[b].at[j, 0]],
                                      rv_[b].at[pl.ds(j * 128, 128)],
                                      gsem[b]).wait()

        def issue_scatters(b):
            for j in range(K):
                pltpu.async_copy(rv_[b].at[pl.ds(j * 128, 128)],
                                 out_sh.at[ip[b].at[j, 1]], ssem[b], add=True)

        def wait_scatters(b):
            for j in range(K):
                pltpu.make_async_copy(rv_[b].at[pl.ds(j * 128, 128)],
                                      out_sh.at[ip[b].at[j, 1]],
                                      ssem[b]).wait()

        def compute(b):
            if gat:
                for j in range(K):
                    for i in range(8):
                        sl16 = pl.ds(i * 16, 16)
                        dv = plsc.load_gather(den0_v, [ip[b][j, 1, sl16]])
                        wv[b][j, sl16] = wv[b][j, sl16] * dv

            def sbody(q, carry2):
                ridx = q * 16 + iota
                jq = ridx // 128
                iq = lax.rem(ridx, 128)
                wq = plsc.load_gather(wv[b], [jq, iq])
                for col in range(FP):
                    cc = jnp.full((16,), col, jnp.int32)
                    rr = plsc.load_gather(rv_[b], [ridx, cc])
                    plsc.store_scatter(rv_[b], [ridx, cc], rr * wq)
                return carry2

            lax.fori_loop(0, EB // 16, sbody, 0)

        issue_inputs(0, 0)
        wait_inputs(0)
        issue_gathers(0)

        def obody(go, carry):
            for b in range(2):
                g = go * 2 + b
                nb = 1 - b

                @pl.when(g + 1 < groups)
                def _():
                    @pl.when(g >= 1)
                    def _():
                        wait_scatters(nb)

                    issue_inputs(nb, g + 1)

                wait_gathers(b)
                compute(b)
                issue_scatters(b)

                @pl.when(g + 1 < groups)
                def _():
                    wait_inputs(nb)
                    issue_gathers(nb)

            return carry

        lax.fori_loop(0, groups // 2, obody, 0)
        wait_scatters((groups - 2) % 2)
        wait_scatters((groups - 1) % 2)
        plsc.subcore_barrier()
        pltpu.sync_copy(out_sh.at[pl.ds(s * zlen, zlen)],
                        out_hbm.at[c, pl.ds(s * zlen, zlen)])

    return rker


_BS = 1000


def _full(shp):
    return pl.BlockSpec(shp, lambda i: (0,) * len(shp))


@functools.lru_cache(maxsize=None)
def _build_affine_first(n, din):
    nb = n // _BS

    def body(x_ref, W_ref, b_ref, pW_ref, pb_ref, h_o, hp_o):
        x = x_ref[...]
        h_o[...] = jnp.dot(x, W_ref[...],
                           preferred_element_type=jnp.float32) + b_ref[...]
        hp_o[...] = jnp.dot(x, pW_ref[...],
                            preferred_element_type=jnp.float32) + pb_ref[...]

    in_specs = [pl.BlockSpec((_BS, din), lambda i: (i, 0)),
                _full((din, FP)), _full((1, FP)), _full((din, FP)),
                _full((1, FP))]
    out_specs = [pl.BlockSpec((_BS, FP), lambda i: (i, 0)),
                 pl.BlockSpec((_BS, FP), lambda i: (i, 0))]
    out = [jax.ShapeDtypeStruct((n, FP), jnp.float32),
           jax.ShapeDtypeStruct((n, FP), jnp.float32)]
    return pl.pallas_call(body, grid=(nb,), in_specs=in_specs,
                          out_specs=out_specs, out_shape=out)


@functools.lru_cache(maxsize=None)
def _build_affine_next(n, n_pad, nparts):
    nb = n // _BS

    def body(*refs):
        parts = refs[:nparts]
        W_ref, b_ref, pW_ref, pb_ref = refs[nparts:nparts + 4]
        x_o, h_o, hp_o = refs[nparts + 4:]
        acc = parts[0][0] + parts[0][1]
        for p in parts[1:]:
            acc = acc + p[0] + p[1]
        x = jnp.maximum(acc, 0.0)
        x_o[...] = x
        h_o[...] = jnp.dot(x, W_ref[...],
                           preferred_element_type=jnp.float32) + b_ref[...]
        hp_o[...] = jnp.dot(x, pW_ref[...],
                            preferred_element_type=jnp.float32) + pb_ref[...]

    part_spec = pl.BlockSpec((NC, _BS, FP), lambda i: (0, i, 0))
    in_specs = [part_spec] * nparts + [
        _full((FP, FP)), _full((1, FP)), _full((FP, FP)), _full((1, FP))]
    out_specs = [pl.BlockSpec((_BS, FP), lambda i: (i, 0))] * 3
    out = [jax.ShapeDtypeStruct((n, FP), jnp.float32),
           jax.ShapeDtypeStruct((n, FP), jnp.float32),
           jax.ShapeDtypeStruct((n, FP), jnp.float32)]
    return pl.pallas_call(body, grid=(nb,), in_specs=in_specs,
                          out_specs=out_specs, out_shape=out)


@functools.lru_cache(maxsize=None)
def _build_attn(n):
    def body(h_ref, asrc_ref, adst_ref, as_o, ad_o, c_o):
        h = h_ref[...]
        a_s = (h @ asrc_ref[0])[None, :]
        a_d = (h @ adst_ref[0])[None, :]
        ms = jnp.max(a_s)
        t = ms + a_d
        as_o[...] = a_s
        ad_o[...] = a_d
        c_o[...] = jnp.maximum(t, 0.2 * t)

    out = [jax.ShapeDtypeStruct((1, n), jnp.float32)] * 3
    return pl.pallas_call(body, out_shape=out)


@functools.lru_cache(maxsize=None)
def _build_dinv(n, n_pad):
    def body(dp_ref, out_ref):
        d = dp_ref[0, :n] + dp_ref[1, :n]
        out_ref[...] = lax.rsqrt(d + 1e-12)[None, :]

    return pl.pallas_call(body, out_shape=jax.ShapeDtypeStruct((1, n), jnp.float32))


@functools.lru_cache(maxsize=None)
def _build_cvec(n):
    def body(as_ref, ad_ref, c_o):
        ms = jnp.max(as_ref[...])
        t = ms + ad_ref[...]
        c_o[...] = jnp.maximum(t, 0.2 * t)

    return pl.pallas_call(body, out_shape=jax.ShapeDtypeStruct((1, n), jnp.float32))


@functools.lru_cache(maxsize=None)
def _build_head(n, n_pad, nparts):
    nb = n // _BS

    def body(*refs):
        x1_ref, x2_ref = refs[:2]
        parts = refs[2:2 + nparts]
        Wa, Wb, Wc, b4 = refs[2 + nparts:2 + nparts + 4]
        out_ref = refs[-1]
        i = pl.program_id(0)
        acc = parts[0][0] + parts[0][1]
        for p in parts[1:]:
            acc = acc + p[0] + p[1]
        x3 = jnp.maximum(acc, 0.0)
        x4 = (jnp.dot(x1_ref[...], Wa[...], preferred_element_type=jnp.float32)
              + jnp.dot(x2_ref[...], Wb[...], preferred_element_type=jnp.float32)
              + jnp.dot(x3, Wc[...], preferred_element_type=jnp.float32))

        @pl.when(i == 0)
        def _():
            out_ref[...] = jnp.zeros_like(out_ref)

        out_ref[...] += jnp.sum(x4, axis=0, keepdims=True)

        @pl.when(i == nb - 1)
        def _():
            out_ref[...] = out_ref[...] * (1.0 / n) + b4[...]

    part_spec = pl.BlockSpec((NC, _BS, FP), lambda i: (0, i, 0))
    in_specs = [pl.BlockSpec((_BS, FP), lambda i: (i, 0))] * 2 + \
        [part_spec] * nparts + [
        _full((FP, OUTD)), _full((FP, OUTD)), _full((FP, OUTD)),
        _full((1, OUTD))]
    return pl.pallas_call(
        body, grid=(nb,), in_specs=in_specs,
        out_specs=pl.BlockSpec((1, OUTD), lambda i: (0, 0)),
        out_shape=jax.ShapeDtypeStruct((1, OUTD), jnp.float32))


def _final_combine(y0, y1, y2, Wc, bc):
    def body(y0_ref, y1_ref, y2_ref, w0, w1, w2, b_ref, out_ref):
        y = (jnp.dot(y0_ref[...], w0[...], preferred_element_type=jnp.float32)
             + jnp.dot(y1_ref[...], w1[...], preferred_element_type=jnp.float32)
             + jnp.dot(y2_ref[...], w2[...], preferred_element_type=jnp.float32)
             + b_ref[...])
        z = y - jnp.max(y)
        e = jnp.exp(z)
        out_ref[...] = e / jnp.sum(e)

    return pl.pallas_call(
        body, out_shape=jax.ShapeDtypeStruct((1, OUTD), jnp.float32))(
        y0, y1, y2, Wc[0:10], Wc[10:20], Wc[20:30], bc[None, :])


def _pad_edge_arr(a, e_pad, dtype):
    E = a.shape[0]
    a = jnp.pad(a.astype(dtype), (0, e_pad - E))
    return a.reshape(e_pad // 128, 128)


def _pad_w(w, fout=FP):
    return jnp.pad(w, ((0, 0), (0, fout - w.shape[1])))


def _pad_w2(w):
    return jnp.pad(w, ((0, FP - w.shape[0]), (0, FP - w.shape[1])))


def _pad_v(v):
    return jnp.pad(v, (0, FP - v.shape[0]))


def _san_weights(p, first):
    ld = p["l_d"]
    W = _pad_w(ld["W"]) if first else _pad_w2(ld["W"])
    pW = _pad_w(p["p_W"]) if first else _pad_w2(p["p_W"])
    return (W, _pad_v(ld["b"])[None, :], pW, _pad_v(p["p_b"])[None, :],
            _pad_v(ld["a_src"])[None, :], _pad_v(ld["a_dst"])[None, :])


def _run_level(X, params, lvl, gat_idx_list, p_idx, p_val, n):
    n_pad = _n_pad(n)
    ep_p = _e_pad(p_idx.shape[1])
    ppack = jnp.stack([_pad_edge_arr(p_idx[1], ep_p, jnp.int32),
                       _pad_edge_arr(p_idx[0], ep_p, jnp.int32)], axis=1)
    pval = _pad_edge_arr(p_val, ep_p, jnp.float32)
    gats = []
    for gi in gat_idx_list:
        ep = _e_pad(gi.shape[1])
        gpack = jnp.stack([_pad_edge_arr(gi[0], ep, jnp.int32),
                           _pad_edge_arr(gi[1], ep, jnp.int32)], axis=1)
        gats.append((gpack, ep, gi.shape[1]))

    dpart = _build_dseg(ep_p, n_pad)(ppack, pval)

    xs = []
    pw = None
    h = hp = a_s = a_d = cvec = None
    for k in (1, 2, 3):
        p = params["l%d_%d" % (lvl, k)]
        if k == 1:
            W, b, pW, pb, asrc, adst = _san_weights(p, True)
            h, hp = _build_affine_first(n, X.shape[1])(X, W, b, pW, pb)
            a_s, a_d, cvec = _build_attn(n)(h, asrc, adst)
            dinv = _build_dinv(n, n_pad)(dpart)
            pw = _build_pvals(ep_p, n)(ppack, pval, dinv.reshape(n))
        parts = [_build_rows(ep_p, n, n_pad, False)(hp, ppack, pw)]
        for (gpack, ep, e_real) in gats:
            ex, den = _build_softmax(ep, e_real, n, n_pad)(
                gpack, a_s.reshape(n), a_d.reshape(n), cvec.reshape(n))
            parts.append(_build_rows(ep, n, n_pad, True)(h, gpack, ex, den))
        if k < 3:
            p2 = params["l%d_%d" % (lvl, k + 1)]
            W, b, pW, pb, asrc, adst = _san_weights(p2, False)
            x, h, hp = _build_affine_next(n, n_pad, len(parts))(
                *parts, W, b, pW, pb)
            a_s, a_d, cvec = _build_attn(n)(h, asrc, adst)
            xs.append(x)
        else:
            lin = params["l%d_4" % lvl]
            W4 = lin["W"]
            Wa = jnp.pad(W4[0:30], ((0, 2), (0, 0)))
            Wb = jnp.pad(W4[30:60], ((0, 2), (0, 0)))
            Wc = jnp.pad(W4[60:90], ((0, 2), (0, 0)))
            return _build_head(n, n_pad, len(parts))(
                xs[0], xs[1], *parts, Wa, Wb, Wc, lin["b"][None, :])


def kernel(X0, X1, X2, L0_idx, L0_val, L1u_idx, L1u_val, L1d_idx, L1d_val,
           L2_idx, L2_val, batch0, batch1, batch2, params):
    L1_idx = jnp.concatenate([L1u_idx, L1d_idx], axis=1)
    L1_val = jnp.concatenate([L1u_val, L1d_val], axis=0)
    y0 = _run_level(X0, params, 0, [L0_idx], L0_idx, L0_val, X0.shape[0])
    y1 = _run_level(X1, params, 1, [L1u_idx, L1d_idx], L1_idx, L1_val, X1.shape[0])
    y2 = _run_level(X2, params, 2, [L2_idx], L2_idx, L2_val, X2.shape[0])
    comb = params["combined"]
    return _final_combine(y0, y1, y2, comb["W"], comb["b"])


# speedup vs baseline: 15.3735x; 1.4062x over previous
import functools

import jax
import jax.numpy as jnp
from jax import lax
from jax.experimental import pallas as pl
from jax.experimental.pallas import tpu as pltpu
from jax.experimental.pallas import tpu_sc as plsc

F = 30
FP = 32
OUTD = 10
NC = 2
NS = 16
NW = NC * NS
K = 4
EBLK = 2 * NW * K * 128

_mesh = plsc.VectorSubcoreMesh(
    core_axis_name="c", subcore_axis_name="s", num_cores=NC, num_subcores=NS)


def _e_pad(e):
    return ((e + EBLK - 1) // EBLK) * EBLK


def _n_pad(n):
    return ((n + 255) // 256) * 256


def _worker_id():
    return lax.axis_index("c") * NS + lax.axis_index("s")


def _zero_vmem(v):
    def zb(i, carry):
        v[pl.ds(i * 16, 16)] = jnp.zeros((16,), jnp.float32)
        return carry

    lax.fori_loop(0, v.shape[0] // 16, zb, 0)


def _reduce_tiles(acc_ref, tmp_v, dall_sh, out_hbm, c, s, zlen):
    pltpu.sync_copy(acc_ref, dall_sh.at[s])
    plsc.subcore_barrier()
    pltpu.sync_copy(dall_sh.at[0, pl.ds(s * zlen, zlen)],
                    acc_ref.at[pl.ds(0, zlen)])
    for t in range(1, NS):
        pltpu.sync_copy(dall_sh.at[t, pl.ds(s * zlen, zlen)], tmp_v)

        def ab(i, carry):
            sl = pl.ds(i * 16, 16)
            acc_ref[sl] = acc_ref[sl] + tmp_v[sl]
            return carry

        lax.fori_loop(0, zlen // 16, ab, 0)
    pltpu.sync_copy(acc_ref.at[pl.ds(0, zlen)],
                    out_hbm.at[c, pl.ds(s * zlen, zlen)])


def _zero_shared(z_v, sh, rows_per_tile):
    s = lax.axis_index("s")
    if len(z_v.shape) == 1:
        for i in range(z_v.shape[0] // 16):
            z_v[pl.ds(i * 16, 16)] = jnp.zeros((16,), jnp.float32)
    else:
        for r in range(z_v.shape[0]):
            for i in range(z_v.shape[1] // 16):
                z_v[r, pl.ds(i * 16, 16)] = jnp.zeros((16,), jnp.float32)

    nz = rows_per_tile // z_v.shape[0]

    def zb(i, carry):
        pltpu.sync_copy(z_v, sh.at[pl.ds(s * rows_per_tile + i * z_v.shape[0],
                                         z_v.shape[0])])
        return carry

    lax.fori_loop(0, nz, zb, 0)


@functools.lru_cache(maxsize=None)
def _build_dseg(e_pad, n_pad):
    rows = e_pad // 128
    rows_w = rows // NW
    groups = rows_w // K
    zlen = n_pad // NS

    @functools.partial(
        pl.kernel,
        out_type=jax.ShapeDtypeStruct((NC, n_pad), jnp.float32),
        mesh=_mesh,
        compiler_params=pltpu.CompilerParams(
            needs_layout_passes=False, use_tc_tiling_on_sc=False),
        scratch_types=[
            pltpu.VMEM((K, 2, 128), jnp.int32),
            pltpu.VMEM((K, 128), jnp.float32),
            pltpu.VMEM((n_pad,), jnp.float32),
            pltpu.VMEM((zlen,), jnp.float32),
            pltpu.VMEM_SHARED((NS, n_pad), jnp.float32),
        ],
    )
    def dker(pack_hbm, val_hbm, dpart_hbm, idx_v, w_v, d_v, tmp_v, dall_sh):
        c = lax.axis_index("c")
        s = lax.axis_index("s")
        wid = _worker_id()
        _zero_vmem(d_v)
        base = wid * rows_w

        def gbody(g, carry):
            r0 = base + g * K
            pltpu.sync_copy(pack_hbm.at[pl.ds(r0, K)], idx_v)
            pltpu.sync_copy(val_hbm.at[pl.ds(r0, K)], w_v)
            for j in range(K):
                for i in range(8):
                    sl16 = pl.ds(i * 16, 16)
                    plsc.addupdate_scatter(
                        d_v, [idx_v[j, 1, sl16]], jnp.abs(w_v[j, sl16]))
            return carry

        lax.fori_loop(0, groups, gbody, 0)
        _reduce_tiles(d_v, tmp_v, dall_sh, dpart_hbm, c, s, zlen)

    return dker


@functools.lru_cache(maxsize=None)
def _build_pvals(e_pad, n):
    rows = e_pad // 128
    rows_w = rows // NW
    groups = rows_w // K

    @functools.partial(
        pl.kernel,
        out_type=jax.ShapeDtypeStruct((rows, 128), jnp.float32),
        mesh=_mesh,
        compiler_params=pltpu.CompilerParams(
            needs_layout_passes=False, use_tc_tiling_on_sc=False),
        scratch_types=[
            pltpu.VMEM((n,), jnp.float32),
            pltpu.VMEM((K, 2, 128), jnp.int32),
            pltpu.VMEM((K, 128), jnp.float32),
        ],
    )
    def pker(pack_hbm, val_hbm, dinv_hbm, w_hbm, dinv_v, idx_v, v_v):
        wid = _worker_id()
        pltpu.sync_copy(dinv_hbm, dinv_v)
        base = wid * rows_w

        def gbody(g, carry):
            r0 = base + g * K
            pltpu.sync_copy(pack_hbm.at[pl.ds(r0, K)], idx_v)
            pltpu.sync_copy(val_hbm.at[pl.ds(r0, K)], v_v)
            for j in range(K):
                for i in range(8):
                    sl16 = pl.ds(i * 16, 16)
                    rv = plsc.load_gather(dinv_v, [idx_v[j, 1, sl16]])
                    cv = plsc.load_gather(dinv_v, [idx_v[j, 0, sl16]])
                    v_v[j, sl16] = v_v[j, sl16] * rv * cv
            pltpu.sync_copy(v_v, w_hbm.at[pl.ds(r0, K)])
            return carry

        lax.fori_loop(0, groups, gbody, 0)

    return pker


@functools.lru_cache(maxsize=None)
def _build_softmax(e_pad, e_real, n, n_pad):
    rows = e_pad // 128
    rows_w = rows // NW
    groups = rows_w // K
    zlen = n_pad // NS

    @functools.partial(
        pl.kernel,
        out_type=(jax.ShapeDtypeStruct((rows, 128), jnp.float32),
                  jax.ShapeDtypeStruct((NC, n_pad), jnp.float32)),
        mesh=_mesh,
        compiler_params=pltpu.CompilerParams(
            needs_layout_passes=False, use_tc_tiling_on_sc=False),
        scratch_types=[
            pltpu.VMEM((n,), jnp.float32),
            pltpu.VMEM((n,), jnp.float32),
            pltpu.VMEM((n,), jnp.float32),
            pltpu.VMEM((K, 2, 128), jnp.int32),
            pltpu.VMEM((K, 128), jnp.float32),
            pltpu.VMEM((n_pad,), jnp.float32),
            pltpu.VMEM((zlen,), jnp.float32),
            pltpu.VMEM_SHARED((NS, n_pad), jnp.float32),
        ],
    )
    def smker(pack_hbm, as_hbm, ad_hbm, c_hbm, ex_hbm, den_hbm,
              as_v, ad_v, c_v, idx_v, ex_v, den_v, tmp_v, dall_sh):
        c = lax.axis_index("c")
        s = lax.axis_index("s")
        wid = _worker_id()
        pltpu.sync_copy(as_hbm, as_v)
        pltpu.sync_copy(ad_hbm, ad_v)
        pltpu.sync_copy(c_hbm, c_v)
        _zero_vmem(den_v)
        base = wid * rows_w
        iota = jnp.arange(16, dtype=jnp.int32)

        def gbody(g, carry):
            r0 = base + g * K
            pltpu.sync_copy(pack_hbm.at[pl.ds(r0, K)], idx_v)
            for j in range(K):
                for i in range(8):
                    sl16 = pl.ds(i * 16, 16)
                    di = idx_v[j, 1, sl16]
                    sv = plsc.load_gather(as_v, [idx_v[j, 0, sl16]])
                    dv = plsc.load_gather(ad_v, [di])
                    cv = plsc.load_gather(c_v, [di])
                    t = sv + dv
                    e = jnp.maximum(t, 0.2 * t)
                    ex = jnp.exp(e - cv)
                    eid = (r0 + j) * 128 + (i * 16) + iota
                    ex = jnp.where(eid < e_real, ex, 0.0)
                    ex_v[j, sl16] = ex
                    plsc.addupdate_scatter(den_v, [di], ex)
            pltpu.sync_copy(ex_v, ex_hbm.at[pl.ds(r0, K)])
            return carry

        lax.fori_loop(0, groups, gbody, 0)
        _reduce_tiles(den_v, tmp_v, dall_sh, den_hbm, c, s, zlen)

    return smker


@functools.lru_cache(maxsize=None)
def _build_rows(e_pad, n, n_pad, gat):
    rows = e_pad // 128
    rows_w = rows // NW
    groups = rows_w // K
    zlen = n_pad // NS
    EB = K * 128

    scratch = [
        pltpu.VMEM((K, 2, 128), jnp.int32),
        pltpu.VMEM((K, 2, 128), jnp.int32),
        pltpu.VMEM((K, 128), jnp.float32),
        pltpu.VMEM((K, 128), jnp.float32),
        pltpu.VMEM((EB, FP), jnp.float32),
        pltpu.VMEM((EB, FP), jnp.float32),
        pltpu.VMEM((EB, FP), jnp.float32),
        pltpu.VMEM((8, FP), jnp.float32),
        pltpu.VMEM_SHARED((n_pad, FP), jnp.float32),
        pltpu.SemaphoreType.DMA,
        pltpu.SemaphoreType.DMA,
        pltpu.SemaphoreType.DMA,
        pltpu.SemaphoreType.DMA,
        pltpu.SemaphoreType.DMA,
        pltpu.SemaphoreType.DMA,
    ]
    if gat:
        scratch += [pltpu.VMEM((n_pad,), jnp.float32),
                    pltpu.VMEM((256,), jnp.float32)]

    @functools.partial(
        pl.kernel,
        out_type=jax.ShapeDtypeStruct((NC, n_pad, FP), jnp.float32),
        mesh=_mesh,
        compiler_params=pltpu.CompilerParams(
            needs_layout_passes=False, use_tc_tiling_on_sc=False),
        scratch_types=scratch,
    )
    def rker(*refs):
        if gat:
            (t_hbm, pack_hbm, w_hbm, den_hbm, out_hbm,
             ip0, ip1, w0, w1, r0_, r1_, ro, z_v, out_sh,
             is0, is1, gs0, gs1, ss0, ss1, den0_v, dtmp_v) = refs
        else:
            (t_hbm, pack_hbm, w_hbm, out_hbm,
             ip0, ip1, w0, w1, r0_, r1_, ro, z_v, out_sh,
             is0, is1, gs0, gs1, ss0, ss1) = refs
        ip = (ip0, ip1)
        wv = (w0, w1)
        rv_ = (r0_, r1_)
        isem = (is0, is1)
        gsem = (gs0, gs1)
        ssem = (ss0, ss1)
        c = lax.axis_index("c")
        s = lax.axis_index("s")
        wid = _worker_id()
        _zero_shared(z_v, out_sh, zlen)
        if gat:
            pltpu.sync_copy(den_hbm.at[0], den0_v)

            def dch(k, carry):
                pltpu.sync_copy(den_hbm.at[1, pl.ds(k * 256, 256)], dtmp_v)

                def ab(i, c2):
                    sl = pl.ds(k * 256 + i * 16, 16)
                    sl2 = pl.ds(i * 16, 16)
                    den0_v[sl] = 1.0 / (den0_v[sl] + dtmp_v[sl2] + 1e-16)
                    return c2

                lax.fori_loop(0, 16, ab, 0)
                return carry

            lax.fori_loop(0, n_pad // 256, dch, 0)
        plsc.subcore_barrier()
        base = wid * rows_w
        iota = jnp.arange(16, dtype=jnp.int32)

        def issue_inputs(b, g):
            r0 = base + g * K
            pltpu.async_copy(pack_hbm.at[pl.ds(r0, K)], ip[b], isem[b])
            pltpu.async_copy(w_hbm.at[pl.ds(r0, K)], wv[b], isem[b])

        def wait_inputs(b):
            pltpu.make_async_copy(pack_hbm.at[pl.ds(0, K)], ip[b],
                                  isem[b]).wait()
            pltpu.make_async_copy(w_hbm.at[pl.ds(0, K)], wv[b],
                                  isem[b]).wait()

        def issue_gathers(b):
            for j in range(K):
                pltpu.async_copy(t_hbm.at[ip[b].at[j, 0]],
                                 rv_[b].at[pl.ds(j * 128, 128)], gsem[b])

        def wait_gathers(b):
            for j in range(K):
                pltpu.make_async_copy(t_hbm.at[ip[b].at[j, 0]],
                                      rv_[b].at[pl.ds(j * 128, 128)],
                                      gsem[b]).wait()

        def issue_scatters(b):
            for j in range(K):
                pltpu.async_copy(ro.at[pl.ds(j * 128, 128)],
                                 out_sh.at[ip[b].at[j, 1]], ssem[b], add=True)

        def wait_scatters(b):
            for j in range(K):
                pltpu.make_async_copy(ro.at[pl.ds(j * 128, 128)],
                                      out_sh.at[ip[b].at[j, 1]],
                                      ssem[b]).wait()

        one16 = jnp.full((16,), 1, jnp.int32)

        def compute(b):
            @plsc.parallel_loop(0, EB // 16, 1, unroll=2)
            def _(q):
                ridx = q * 16 + iota
                jq = ridx // 128
                iq = lax.rem(ridx, 128)
                wq = plsc.load_gather(wv[b], [jq, iq])
                if gat:
                    di = plsc.load_gather(ip[b], [jq, one16, iq])
                    wq = wq * plsc.load_gather(den0_v, [di])
                for col in range(FP):
                    cc = jnp.full((16,), col, jnp.int32)
                    rr = plsc.load_gather(rv_[b], [ridx, cc])
                    plsc.store_scatter(ro, [ridx, cc], rr * wq)

        issue_inputs(0, 0)
        wait_inputs(0)
        issue_gathers(0)

        def obody(go, carry):
            for b in range(2):
                g = go * 2 + b
                nb = 1 - b

                @pl.when(g >= 1)
                def _():
                    wait_scatters(nb)

                @pl.when(g + 1 < groups)
                def _():
                    issue_inputs(nb, g + 1)

                wait_gathers(b)
                compute(b)
                issue_scatters(b)

                @pl.when(g + 1 < groups)
                def _():
                    wait_inputs(nb)
                    issue_gathers(nb)

            return carry

        lax.fori_loop(0, groups // 2, obody, 0)
        wait_scatters((groups - 1) % 2)
        plsc.subcore_barrier()
        pltpu.sync_copy(out_sh.at[pl.ds(s * zlen, zlen)],
                        out_hbm.at[c, pl.ds(s * zlen, zlen)])

    return rker


_BS = 1000


def _full(shp):
    return pl.BlockSpec(shp, lambda i: (0,) * len(shp))


@functools.lru_cache(maxsize=None)
def _build_affine_first(n, din):
    nb = n // _BS

    def body(x_ref, W_ref, b_ref, pW_ref, pb_ref, h_o, hp_o):
        x = x_ref[...]
        h_o[...] = jnp.dot(x, W_ref[...],
                           preferred_element_type=jnp.float32) + b_ref[...]
        hp_o[...] = jnp.dot(x, pW_ref[...],
                            preferred_element_type=jnp.float32) + pb_ref[...]

    in_specs = [pl.BlockSpec((_BS, din), lambda i: (i, 0)),
                _full((din, FP)), _full((1, FP)), _full((din, FP)),
                _full((1, FP))]
    out_specs = [pl.BlockSpec((_BS, FP), lambda i: (i, 0)),
                 pl.BlockSpec((_BS, FP), lambda i: (i, 0))]
    out = [jax.ShapeDtypeStruct((n, FP), jnp.float32),
           jax.ShapeDtypeStruct((n, FP), jnp.float32)]
    return pl.pallas_call(body, grid=(nb,), in_specs=in_specs,
                          out_specs=out_specs, out_shape=out)


@functools.lru_cache(maxsize=None)
def _build_affine_next(n, n_pad, nparts):
    nb = n // _BS

    def body(*refs):
        parts = refs[:nparts]
        W_ref, b_ref, pW_ref, pb_ref = refs[nparts:nparts + 4]
        x_o, h_o, hp_o = refs[nparts + 4:]
        acc = parts[0][0] + parts[0][1]
        for p in parts[1:]:
            acc = acc + p[0] + p[1]
        x = jnp.maximum(acc, 0.0)
        x_o[...] = x
        h_o[...] = jnp.dot(x, W_ref[...],
                           preferred_element_type=jnp.float32) + b_ref[...]
        hp_o[...] = jnp.dot(x, pW_ref[...],
                            preferred_element_type=jnp.float32) + pb_ref[...]

    part_spec = pl.BlockSpec((NC, _BS, FP), lambda i: (0, i, 0))
    in_specs = [part_spec] * nparts + [
        _full((FP, FP)), _full((1, FP)), _full((FP, FP)), _full((1, FP))]
    out_specs = [pl.BlockSpec((_BS, FP), lambda i: (i, 0))] * 3
    out = [jax.ShapeDtypeStruct((n, FP), jnp.float32),
           jax.ShapeDtypeStruct((n, FP), jnp.float32),
           jax.ShapeDtypeStruct((n, FP), jnp.float32)]
    return pl.pallas_call(body, grid=(nb,), in_specs=in_specs,
                          out_specs=out_specs, out_shape=out)


@functools.lru_cache(maxsize=None)
def _build_attn(n):
    def body(h_ref, asrc_ref, adst_ref, as_o, ad_o, c_o):
        h = h_ref[...]
        a_s = (h @ asrc_ref[0])[None, :]
        a_d = (h @ adst_ref[0])[None, :]
        ms = jnp.max(a_s)
        t = ms + a_d
        as_o[...] = a_s
        ad_o[...] = a_d
        c_o[...] = jnp.maximum(t, 0.2 * t)

    out = [jax.ShapeDtypeStruct((1, n), jnp.float32)] * 3
    return pl.pallas_call(body, out_shape=out)


@functools.lru_cache(maxsize=None)
def _build_dinv(n, n_pad):
    def body(dp_ref, out_ref):
        d = dp_ref[0, :n] + dp_ref[1, :n]
        out_ref[...] = lax.rsqrt(d + 1e-12)[None, :]

    return pl.pallas_call(body, out_shape=jax.ShapeDtypeStruct((1, n), jnp.float32))


@functools.lru_cache(maxsize=None)
def _build_cvec(n):
    def body(as_ref, ad_ref, c_o):
        ms = jnp.max(as_ref[...])
        t = ms + ad_ref[...]
        c_o[...] = jnp.maximum(t, 0.2 * t)

    return pl.pallas_call(body, out_shape=jax.ShapeDtypeStruct((1, n), jnp.float32))


@functools.lru_cache(maxsize=None)
def _build_head(n, n_pad, nparts):
    nb = n // _BS

    def body(*refs):
        x1_ref, x2_ref = refs[:2]
        parts = refs[2:2 + nparts]
        Wa, Wb, Wc, b4 = refs[2 + nparts:2 + nparts + 4]
        out_ref = refs[-1]
        i = pl.program_id(0)
        acc = parts[0][0] + parts[0][1]
        for p in parts[1:]:
            acc = acc + p[0] + p[1]
        x3 = jnp.maximum(acc, 0.0)
        x4 = (jnp.dot(x1_ref[...], Wa[...], preferred_element_type=jnp.float32)
              + jnp.dot(x2_ref[...], Wb[...], preferred_element_type=jnp.float32)
              + jnp.dot(x3, Wc[...], preferred_element_type=jnp.float32))

        @pl.when(i == 0)
        def _():
            out_ref[...] = jnp.zeros_like(out_ref)

        out_ref[...] += jnp.sum(x4, axis=0, keepdims=True)

        @pl.when(i == nb - 1)
        def _():
            out_ref[...] = out_ref[...] * (1.0 / n) + b4[...]

    part_spec = pl.BlockSpec((NC, _BS, FP), lambda i: (0, i, 0))
    in_specs = [pl.BlockSpec((_BS, FP), lambda i: (i, 0))] * 2 + \
        [part_spec] * nparts + [
        _full((FP, OUTD)), _full((FP, OUTD)), _full((FP, OUTD)),
        _full((1, OUTD))]
    return pl.pallas_call(
        body, grid=(nb,), in_specs=in_specs,
        out_specs=pl.BlockSpec((1, OUTD), lambda i: (0, 0)),
        out_shape=jax.ShapeDtypeStruct((1, OUTD), jnp.float32))


def _final_combine(y0, y1, y2, Wc, bc):
    def body(y0_ref, y1_ref, y2_ref, w0, w1, w2, b_ref, out_ref):
        y = (jnp.dot(y0_ref[...], w0[...], preferred_element_type=jnp.float32)
             + jnp.dot(y1_ref[...], w1[...], preferred_element_type=jnp.float32)
             + jnp.dot(y2_ref[...], w2[...], preferred_element_type=jnp.float32)
             + b_ref[...])
        z = y - jnp.max(y)
        e = jnp.exp(z)
        out_ref[...] = e / jnp.sum(e)

    return pl.pallas_call(
        body, out_shape=jax.ShapeDtypeStruct((1, OUTD), jnp.float32))(
        y0, y1, y2, Wc[0:10], Wc[10:20], Wc[20:30], bc[None, :])


def _pad_edge_arr(a, e_pad, dtype):
    E = a.shape[0]
    a = jnp.pad(a.astype(dtype), (0, e_pad - E))
    return a.reshape(e_pad // 128, 128)


def _pad_w(w, fout=FP):
    return jnp.pad(w, ((0, 0), (0, fout - w.shape[1])))


def _pad_w2(w):
    return jnp.pad(w, ((0, FP - w.shape[0]), (0, FP - w.shape[1])))


def _pad_v(v):
    return jnp.pad(v, (0, FP - v.shape[0]))


def _san_weights(p, first):
    ld = p["l_d"]
    W = _pad_w(ld["W"]) if first else _pad_w2(ld["W"])
    pW = _pad_w(p["p_W"]) if first else _pad_w2(p["p_W"])
    return (W, _pad_v(ld["b"])[None, :], pW, _pad_v(p["p_b"])[None, :],
            _pad_v(ld["a_src"])[None, :], _pad_v(ld["a_dst"])[None, :])


def _run_level(X, params, lvl, gat_idx_list, p_idx, p_val, n):
    n_pad = _n_pad(n)
    ep_p = _e_pad(p_idx.shape[1])
    ppack = jnp.stack([_pad_edge_arr(p_idx[1], ep_p, jnp.int32),
                       _pad_edge_arr(p_idx[0], ep_p, jnp.int32)], axis=1)
    pval = _pad_edge_arr(p_val, ep_p, jnp.float32)
    gats = []
    for gi in gat_idx_list:
        ep = _e_pad(gi.shape[1])
        gpack = jnp.stack([_pad_edge_arr(gi[0], ep, jnp.int32),
                           _pad_edge_arr(gi[1], ep, jnp.int32)], axis=1)
        gats.append((gpack, ep, gi.shape[1]))

    dpart = _build_dseg(ep_p, n_pad)(ppack, pval)

    xs = []
    pw = None
    h = hp = a_s = a_d = cvec = None
    for k in (1, 2, 3):
        p = params["l%d_%d" % (lvl, k)]
        if k == 1:
            W, b, pW, pb, asrc, adst = _san_weights(p, True)
            h, hp = _build_affine_first(n, X.shape[1])(X, W, b, pW, pb)
            a_s, a_d, cvec = _build_attn(n)(h, asrc, adst)
            dinv = _build_dinv(n, n_pad)(dpart)
            pw = _build_pvals(ep_p, n)(ppack, pval, dinv.reshape(n))
        parts = [_build_rows(ep_p, n, n_pad, False)(hp, ppack, pw)]
        for (gpack, ep, e_real) in gats:
            ex, den = _build_softmax(ep, e_real, n, n_pad)(
                gpack, a_s.reshape(n), a_d.reshape(n), cvec.reshape(n))
            parts.append(_build_rows(ep, n, n_pad, True)(h, gpack, ex, den))
        if k < 3:
            p2 = params["l%d_%d" % (lvl, k + 1)]
            W, b, pW, pb, asrc, adst = _san_weights(p2, False)
            x, h, hp = _build_affine_next(n, n_pad, len(parts))(
                *parts, W, b, pW, pb)
            a_s, a_d, cvec = _build_attn(n)(h, asrc, adst)
            xs.append(x)
        else:
            lin = params["l%d_4" % lvl]
            W4 = lin["W"]
            Wa = jnp.pad(W4[0:30], ((0, 2), (0, 0)))
            Wb = jnp.pad(W4[30:60], ((0, 2), (0, 0)))
            Wc = jnp.pad(W4[60:90], ((0, 2), (0, 0)))
            return _build_head(n, n_pad, len(parts))(
                xs[0], xs[1], *parts, Wa, Wb, Wc, lin["b"][None, :])


def kernel(X0, X1, X2, L0_idx, L0_val, L1u_idx, L1u_val, L1d_idx, L1d_val,
           L2_idx, L2_val, batch0, batch1, batch2, params):
    L1_idx = jnp.concatenate([L1u_idx, L1d_idx], axis=1)
    L1_val = jnp.concatenate([L1u_val, L1d_val], axis=0)
    y0 = _run_level(X0, params, 0, [L0_idx], L0_idx, L0_val, X0.shape[0])
    y1 = _run_level(X1, params, 1, [L1u_idx, L1d_idx], L1_idx, L1_val, X1.shape[0])
    y2 = _run_level(X2, params, 2, [L2_idx], L2_idx, L2_val, X2.shape[0])
    comb = params["combined"]
    return _final_combine(y0, y1, y2, comb["W"], comb["b"])


# speedup vs baseline: 15.8052x; 1.0281x over previous
import functools

import jax
import jax.numpy as jnp
from jax import lax
from jax.experimental import pallas as pl
from jax.experimental.pallas import tpu as pltpu
from jax.experimental.pallas import tpu_sc as plsc

F = 30
FP = 32
OUTD = 10
NC = 2
NS = 16
NW = NC * NS
K = 4
EBLK = 2 * NW * K * 128

_mesh = plsc.VectorSubcoreMesh(
    core_axis_name="c", subcore_axis_name="s", num_cores=NC, num_subcores=NS)


def _e_pad(e):
    return ((e + EBLK - 1) // EBLK) * EBLK


def _n_pad(n):
    return ((n + 255) // 256) * 256


def _worker_id():
    return lax.axis_index("c") * NS + lax.axis_index("s")


def _zero_vmem(v):
    def zb(i, carry):
        v[pl.ds(i * 16, 16)] = jnp.zeros((16,), jnp.float32)
        return carry

    lax.fori_loop(0, v.shape[0] // 16, zb, 0)


def _reduce_tiles(acc_ref, tmp_v, dall_sh, out_hbm, c, s, zlen):
    pltpu.sync_copy(acc_ref, dall_sh.at[s])
    plsc.subcore_barrier()
    pltpu.sync_copy(dall_sh.at[0, pl.ds(s * zlen, zlen)],
                    acc_ref.at[pl.ds(0, zlen)])
    for t in range(1, NS):
        pltpu.sync_copy(dall_sh.at[t, pl.ds(s * zlen, zlen)], tmp_v)

        def ab(i, carry):
            sl = pl.ds(i * 16, 16)
            acc_ref[sl] = acc_ref[sl] + tmp_v[sl]
            return carry

        lax.fori_loop(0, zlen // 16, ab, 0)
    pltpu.sync_copy(acc_ref.at[pl.ds(0, zlen)],
                    out_hbm.at[c, pl.ds(s * zlen, zlen)])


def _zero_shared(z_v, sh, rows_per_tile):
    s = lax.axis_index("s")
    if len(z_v.shape) == 1:
        for i in range(z_v.shape[0] // 16):
            z_v[pl.ds(i * 16, 16)] = jnp.zeros((16,), jnp.float32)
    else:
        for r in range(z_v.shape[0]):
            for i in range(z_v.shape[1] // 16):
                z_v[r, pl.ds(i * 16, 16)] = jnp.zeros((16,), jnp.float32)

    nz = rows_per_tile // z_v.shape[0]

    def zb(i, carry):
        pltpu.sync_copy(z_v, sh.at[pl.ds(s * rows_per_tile + i * z_v.shape[0],
                                         z_v.shape[0])])
        return carry

    lax.fori_loop(0, nz, zb, 0)


@functools.lru_cache(maxsize=None)
def _build_dseg(e_pad, n_pad):
    rows = e_pad // 128
    rows_w = rows // NW
    groups = rows_w // K
    zlen = n_pad // NS

    @functools.partial(
        pl.kernel,
        out_type=jax.ShapeDtypeStruct((NC, n_pad), jnp.float32),
        mesh=_mesh,
        compiler_params=pltpu.CompilerParams(
            needs_layout_passes=False, use_tc_tiling_on_sc=False),
        scratch_types=[
            pltpu.VMEM((K, 2, 128), jnp.int32),
            pltpu.VMEM((K, 128), jnp.float32),
            pltpu.VMEM((n_pad,), jnp.float32),
            pltpu.VMEM((zlen,), jnp.float32),
            pltpu.VMEM_SHARED((NS, n_pad), jnp.float32),
        ],
    )
    def dker(pack_hbm, val_hbm, dpart_hbm, idx_v, w_v, d_v, tmp_v, dall_sh):
        c = lax.axis_index("c")
        s = lax.axis_index("s")
        wid = _worker_id()
        _zero_vmem(d_v)
        base = wid * rows_w

        def gbody(g, carry):
            r0 = base + g * K
            pltpu.sync_copy(pack_hbm.at[pl.ds(r0, K)], idx_v)
            pltpu.sync_copy(val_hbm.at[pl.ds(r0, K)], w_v)
            for j in range(K):
                for i in range(8):
                    sl16 = pl.ds(i * 16, 16)
                    plsc.addupdate_scatter(
                        d_v, [idx_v[j, 1, sl16]], jnp.abs(w_v[j, sl16]))
            return carry

        lax.fori_loop(0, groups, gbody, 0)
        _reduce_tiles(d_v, tmp_v, dall_sh, dpart_hbm, c, s, zlen)

    return dker


@functools.lru_cache(maxsize=None)
def _build_pvals(e_pad, n):
    rows = e_pad // 128
    rows_w = rows // NW
    groups = rows_w // K

    @functools.partial(
        pl.kernel,
        out_type=jax.ShapeDtypeStruct((rows, 128), jnp.float32),
        mesh=_mesh,
        compiler_params=pltpu.CompilerParams(
            needs_layout_passes=False, use_tc_tiling_on_sc=False),
        scratch_types=[
            pltpu.VMEM((n,), jnp.float32),
            pltpu.VMEM((K, 2, 128), jnp.int32),
            pltpu.VMEM((K, 128), jnp.float32),
        ],
    )
    def pker(pack_hbm, val_hbm, dinv_hbm, w_hbm, dinv_v, idx_v, v_v):
        wid = _worker_id()
        pltpu.sync_copy(dinv_hbm, dinv_v)
        base = wid * rows_w

        def gbody(g, carry):
            r0 = base + g * K
            pltpu.sync_copy(pack_hbm.at[pl.ds(r0, K)], idx_v)
            pltpu.sync_copy(val_hbm.at[pl.ds(r0, K)], v_v)
            for j in range(K):
                for i in range(8):
                    sl16 = pl.ds(i * 16, 16)
                    rv = plsc.load_gather(dinv_v, [idx_v[j, 1, sl16]])
                    cv = plsc.load_gather(dinv_v, [idx_v[j, 0, sl16]])
                    v_v[j, sl16] = v_v[j, sl16] * rv * cv
            pltpu.sync_copy(v_v, w_hbm.at[pl.ds(r0, K)])
            return carry

        lax.fori_loop(0, groups, gbody, 0)

    return pker


@functools.lru_cache(maxsize=None)
def _build_softmax(e_pad, e_real, n, n_pad):
    rows = e_pad // 128
    rows_w = rows // NW
    groups = rows_w // K
    zlen = n_pad // NS

    @functools.partial(
        pl.kernel,
        out_type=(jax.ShapeDtypeStruct((rows, 128), jnp.float32),
                  jax.ShapeDtypeStruct((NC, n_pad), jnp.float32)),
        mesh=_mesh,
        compiler_params=pltpu.CompilerParams(
            needs_layout_passes=False, use_tc_tiling_on_sc=False),
        scratch_types=[
            pltpu.VMEM((n,), jnp.float32),
            pltpu.VMEM((n,), jnp.float32),
            pltpu.VMEM((n,), jnp.float32),
            pltpu.VMEM((K, 2, 128), jnp.int32),
            pltpu.VMEM((K, 2, 128), jnp.int32),
            pltpu.VMEM((K, 128), jnp.float32),
            pltpu.VMEM((K, 128), jnp.float32),
            pltpu.VMEM((n_pad,), jnp.float32),
            pltpu.VMEM((zlen,), jnp.float32),
            pltpu.VMEM_SHARED((NS, n_pad), jnp.float32),
            pltpu.SemaphoreType.DMA,
            pltpu.SemaphoreType.DMA,
            pltpu.SemaphoreType.DMA,
            pltpu.SemaphoreType.DMA,
        ],
    )
    def smker(pack_hbm, as_hbm, ad_hbm, c_hbm, ex_hbm, den_hbm,
              as_v, ad_v, c_v, ip0, ip1, ex0, ex1, den_v, tmp_v, dall_sh,
              is0, is1, os0, os1):
        ip = (ip0, ip1)
        exv = (ex0, ex1)
        isem = (is0, is1)
        osem = (os0, os1)
        c = lax.axis_index("c")
        s = lax.axis_index("s")
        wid = _worker_id()
        pltpu.sync_copy(as_hbm, as_v)
        pltpu.sync_copy(ad_hbm, ad_v)
        pltpu.sync_copy(c_hbm, c_v)
        _zero_vmem(den_v)
        base = wid * rows_w
        iota = jnp.arange(16, dtype=jnp.int32)
        zero16 = jnp.zeros((16,), jnp.int32)
        one16 = jnp.full((16,), 1, jnp.int32)

        def issue_idx(b, g):
            pltpu.async_copy(pack_hbm.at[pl.ds(base + g * K, K)], ip[b],
                             isem[b])

        def wait_idx(b):
            pltpu.make_async_copy(pack_hbm.at[pl.ds(0, K)], ip[b],
                                  isem[b]).wait()

        def issue_ex(b, g):
            pltpu.async_copy(exv[b], ex_hbm.at[pl.ds(base + g * K, K)],
                             osem[b])

        def wait_ex(b):
            pltpu.make_async_copy(exv[b], ex_hbm.at[pl.ds(0, K)],
                                  osem[b]).wait()

        def compute(b, g):
            r0 = base + g * K

            @plsc.parallel_loop(0, K * 8, 1, unroll=2)
            def _(q):
                pos = q * 16 + iota
                jq = pos // 128
                iq = lax.rem(pos, 128)
                si = plsc.load_gather(ip[b], [jq, zero16, iq])
                di = plsc.load_gather(ip[b], [jq, one16, iq])
                sv = plsc.load_gather(as_v, [si])
                dv = plsc.load_gather(ad_v, [di])
                cv = plsc.load_gather(c_v, [di])
                t = sv + dv
                e = jnp.maximum(t, 0.2 * t)
                ex = jnp.exp(e - cv)
                eid = r0 * 128 + pos
                ex = jnp.where(eid < e_real, ex, 0.0)
                plsc.store_scatter(exv[b], [jq, iq], ex)
                plsc.addupdate_scatter(den_v, [di], ex)

        issue_idx(0, 0)

        def obody(go, carry):
            for b in range(2):
                g = go * 2 + b
                nb = 1 - b

                @pl.when(g + 1 < groups)
                def _():
                    issue_idx(nb, g + 1)

                wait_idx(b)

                @pl.when(g >= 2)
                def _():
                    wait_ex(b)

                compute(b, g)
                issue_ex(b, g)
            return carry

        lax.fori_loop(0, groups // 2, obody, 0)
        wait_ex(0)
        wait_ex(1)
        _reduce_tiles(den_v, tmp_v, dall_sh, den_hbm, c, s, zlen)

    return smker


@functools.lru_cache(maxsize=None)
def _build_rows(e_pad, n, n_pad, gat):
    rows = e_pad // 128
    rows_w = rows // NW
    groups = rows_w // K
    zlen = n_pad // NS
    EB = K * 128

    scratch = [
        pltpu.VMEM((K, 2, 128), jnp.int32),
        pltpu.VMEM((K, 2, 128), jnp.int32),
        pltpu.VMEM((K, 128), jnp.float32),
        pltpu.VMEM((K, 128), jnp.float32),
        pltpu.VMEM((EB, FP), jnp.float32),
        pltpu.VMEM((EB, FP), jnp.float32),
        pltpu.VMEM((EB, FP), jnp.float32),
        pltpu.VMEM((8, FP), jnp.float32),
        pltpu.VMEM_SHARED((n_pad, FP), jnp.float32),
        pltpu.SemaphoreType.DMA,
        pltpu.SemaphoreType.DMA,
        pltpu.SemaphoreType.DMA,
        pltpu.SemaphoreType.DMA,
        pltpu.SemaphoreType.DMA,
        pltpu.SemaphoreType.DMA,
    ]
    if gat:
        scratch += [pltpu.VMEM((n_pad,), jnp.float32),
                    pltpu.VMEM((256,), jnp.float32)]

    @functools.partial(
        pl.kernel,
        out_type=jax.ShapeDtypeStruct((NC, n_pad, FP), jnp.float32),
        mesh=_mesh,
        compiler_params=pltpu.CompilerParams(
            needs_layout_passes=False, use_tc_tiling_on_sc=False),
        scratch_types=scratch,
    )
    def rker(*refs):
        if gat:
            (t_hbm, pack_hbm, w_hbm, den_hbm, out_hbm,
             ip0, ip1, w0, w1, r0_, r1_, ro, z_v, out_sh,
             is0, is1, gs0, gs1, ss0, ss1, den0_v, dtmp_v) = refs
        else:
            (t_hbm, pack_hbm, w_hbm, out_hbm,
             ip0, ip1, w0, w1, r0_, r1_, ro, z_v, out_sh,
             is0, is1, gs0, gs1, ss0, ss1) = refs
        ip = (ip0, ip1)
        wv = (w0, w1)
        rv_ = (r0_, r1_)
        isem = (is0, is1)
        gsem = (gs0, gs1)
        ssem = (ss0, ss1)
        c = lax.axis_index("c")
        s = lax.axis_index("s")
        wid = _worker_id()
        _zero_shared(z_v, out_sh, zlen)
        if gat:
            pltpu.sync_copy(den_hbm.at[0], den0_v)

            def dch(k, carry):
                pltpu.sync_copy(den_hbm.at[1, pl.ds(k * 256, 256)], dtmp_v)

                def ab(i, c2):
                    sl = pl.ds(k * 256 + i * 16, 16)
                    sl2 = pl.ds(i * 16, 16)
                    den0_v[sl] = 1.0 / (den0_v[sl] + dtmp_v[sl2] + 1e-16)
                    return c2

                lax.fori_loop(0, 16, ab, 0)
                return carry

            lax.fori_loop(0, n_pad // 256, dch, 0)
        plsc.subcore_barrier()
        base = wid * rows_w
        iota = jnp.arange(16, dtype=jnp.int32)

        def issue_inputs(b, g):
            r0 = base + g * K
            pltpu.async_copy(pack_hbm.at[pl.ds(r0, K)], ip[b], isem[b])
            pltpu.async_copy(w_hbm.at[pl.ds(r0, K)], wv[b], isem[b])

        def wait_inputs(b):
            pltpu.make_async_copy(pack_hbm.at[pl.ds(0, K)], ip[b],
                                  isem[b]).wait()
            pltpu.make_async_copy(w_hbm.at[pl.ds(0, K)], wv[b],
                                  isem[b]).wait()

        def issue_gathers(b):
            for j in range(K):
                pltpu.async_copy(t_hbm.at[ip[b].at[j, 0]],
                                 rv_[b].at[pl.ds(j * 128, 128)], gsem[b])

        def wait_gathers(b):
            for j in range(K):
                pltpu.make_async_copy(t_hbm.at[ip[b].at[j, 0]],
                                      rv_[b].at[pl.ds(j * 128, 128)],
                                      gsem[b]).wait()

        def issue_scatters(b):
            for j in range(K):
                pltpu.async_copy(ro.at[pl.ds(j * 128, 128)],
                                 out_sh.at[ip[b].at[j, 1]], ssem[b], add=True)

        def wait_scatters(b):
            for j in range(K):
                pltpu.make_async_copy(ro.at[pl.ds(j * 128, 128)],
                                      out_sh.at[ip[b].at[j, 1]],
                                      ssem[b]).wait()

        one16 = jnp.full((16,), 1, jnp.int32)

        def compute(b):
            @plsc.parallel_loop(0, EB // 16, 1, unroll=2)
            def _(q):
                ridx = q * 16 + iota
                jq = ridx // 128
                iq = lax.rem(ridx, 128)
                wq = plsc.load_gather(wv[b], [jq, iq])
                if gat:
                    di = plsc.load_gather(ip[b], [jq, one16, iq])
                    wq = wq * plsc.load_gather(den0_v, [di])
                for col in range(FP):
                    cc = jnp.full((16,), col, jnp.int32)
                    rr = plsc.load_gather(rv_[b], [ridx, cc])
                    plsc.store_scatter(ro, [ridx, cc], rr * wq)

        issue_inputs(0, 0)
        wait_inputs(0)
        issue_gathers(0)

        def obody(go, carry):
            for b in range(2):
                g = go * 2 + b
                nb = 1 - b

                @pl.when(g >= 1)
                def _():
                    wait_scatters(nb)

                @pl.when(g + 1 < groups)
                def _():
                    issue_inputs(nb, g + 1)

                wait_gathers(b)
                compute(b)
                issue_scatters(b)

                @pl.when(g + 1 < groups)
                def _():
                    wait_inputs(nb)
                    issue_gathers(nb)

            return carry

        lax.fori_loop(0, groups // 2, obody, 0)
        wait_scatters((groups - 1) % 2)
        plsc.subcore_barrier()
        pltpu.sync_copy(out_sh.at[pl.ds(s * zlen, zlen)],
                        out_hbm.at[c, pl.ds(s * zlen, zlen)])

    return rker


_BS = 1000


def _full(shp):
    return pl.BlockSpec(shp, lambda i: (0,) * len(shp))


@functools.lru_cache(maxsize=None)
def _build_affine_first(n, din):
    nb = n // _BS

    def body(x_ref, W_ref, b_ref, pW_ref, pb_ref, h_o, hp_o):
        x = x_ref[...]
        h_o[...] = jnp.dot(x, W_ref[...],
                           preferred_element_type=jnp.float32) + b_ref[...]
        hp_o[...] = jnp.dot(x, pW_ref[...],
                            preferred_element_type=jnp.float32) + pb_ref[...]

    in_specs = [pl.BlockSpec((_BS, din), lambda i: (i, 0)),
                _full((din, FP)), _full((1, FP)), _full((din, FP)),
                _full((1, FP))]
    out_specs = [pl.BlockSpec((_BS, FP), lambda i: (i, 0)),
                 pl.BlockSpec((_BS, FP), lambda i: (i, 0))]
    out = [jax.ShapeDtypeStruct((n, FP), jnp.float32),
           jax.ShapeDtypeStruct((n, FP), jnp.float32)]
    return pl.pallas_call(body, grid=(nb,), in_specs=in_specs,
                          out_specs=out_specs, out_shape=out)


@functools.lru_cache(maxsize=None)
def _build_affine_next(n, n_pad, nparts):
    nb = n // _BS

    def body(*refs):
        parts = refs[:nparts]
        W_ref, b_ref, pW_ref, pb_ref = refs[nparts:nparts + 4]
        x_o, h_o, hp_o = refs[nparts + 4:]
        acc = parts[0][0] + parts[0][1]
        for p in parts[1:]:
            acc = acc + p[0] + p[1]
        x = jnp.maximum(acc, 0.0)
        x_o[...] = x
        h_o[...] = jnp.dot(x, W_ref[...],
                           preferred_element_type=jnp.float32) + b_ref[...]
        hp_o[...] = jnp.dot(x, pW_ref[...],
                            preferred_element_type=jnp.float32) + pb_ref[...]

    part_spec = pl.BlockSpec((NC, _BS, FP), lambda i: (0, i, 0))
    in_specs = [part_spec] * nparts + [
        _full((FP, FP)), _full((1, FP)), _full((FP, FP)), _full((1, FP))]
    out_specs = [pl.BlockSpec((_BS, FP), lambda i: (i, 0))] * 3
    out = [jax.ShapeDtypeStruct((n, FP), jnp.float32),
           jax.ShapeDtypeStruct((n, FP), jnp.float32),
           jax.ShapeDtypeStruct((n, FP), jnp.float32)]
    return pl.pallas_call(body, grid=(nb,), in_specs=in_specs,
                          out_specs=out_specs, out_shape=out)


@functools.lru_cache(maxsize=None)
def _build_attn(n):
    def body(h_ref, asrc_ref, adst_ref, as_o, ad_o, c_o):
        h = h_ref[...]
        a_s = (h @ asrc_ref[0])[None, :]
        a_d = (h @ adst_ref[0])[None, :]
        ms = jnp.max(a_s)
        t = ms + a_d
        as_o[...] = a_s
        ad_o[...] = a_d
        c_o[...] = jnp.maximum(t, 0.2 * t)

    out = [jax.ShapeDtypeStruct((1, n), jnp.float32)] * 3
    return pl.pallas_call(body, out_shape=out)


@functools.lru_cache(maxsize=None)
def _build_dinv(n, n_pad):
    def body(dp_ref, out_ref):
        d = dp_ref[0, :n] + dp_ref[1, :n]
        out_ref[...] = lax.rsqrt(d + 1e-12)[None, :]

    return pl.pallas_call(body, out_shape=jax.ShapeDtypeStruct((1, n), jnp.float32))


@functools.lru_cache(maxsize=None)
def _build_cvec(n):
    def body(as_ref, ad_ref, c_o):
        ms = jnp.max(as_ref[...])
        t = ms + ad_ref[...]
        c_o[...] = jnp.maximum(t, 0.2 * t)

    return pl.pallas_call(body, out_shape=jax.ShapeDtypeStruct((1, n), jnp.float32))


@functools.lru_cache(maxsize=None)
def _build_head(n, n_pad, nparts):
    nb = n // _BS

    def body(*refs):
        x1_ref, x2_ref = refs[:2]
        parts = refs[2:2 + nparts]
        Wa, Wb, Wc, b4 = refs[2 + nparts:2 + nparts + 4]
        out_ref = refs[-1]
        i = pl.program_id(0)
        acc = parts[0][0] + parts[0][1]
        for p in parts[1:]:
            acc = acc + p[0] + p[1]
        x3 = jnp.maximum(acc, 0.0)
        x4 = (jnp.dot(x1_ref[...], Wa[...], preferred_element_type=jnp.float32)
              + jnp.dot(x2_ref[...], Wb[...], preferred_element_type=jnp.float32)
              + jnp.dot(x3, Wc[...], preferred_element_type=jnp.float32))

        @pl.when(i == 0)
        def _():
            out_ref[...] = jnp.zeros_like(out_ref)

        out_ref[...] += jnp.sum(x4, axis=0, keepdims=True)

        @pl.when(i == nb - 1)
        def _():
            out_ref[...] = out_ref[...] * (1.0 / n) + b4[...]

    part_spec = pl.BlockSpec((NC, _BS, FP), lambda i: (0, i, 0))
    in_specs = [pl.BlockSpec((_BS, FP), lambda i: (i, 0))] * 2 + \
        [part_spec] * nparts + [
        _full((FP, OUTD)), _full((FP, OUTD)), _full((FP, OUTD)),
        _full((1, OUTD))]
    return pl.pallas_call(
        body, grid=(nb,), in_specs=in_specs,
        out_specs=pl.BlockSpec((1, OUTD), lambda i: (0, 0)),
        out_shape=jax.ShapeDtypeStruct((1, OUTD), jnp.float32))


def _final_combine(y0, y1, y2, Wc, bc):
    def body(y0_ref, y1_ref, y2_ref, w0, w1, w2, b_ref, out_ref):
        y = (jnp.dot(y0_ref[...], w0[...], preferred_element_type=jnp.float32)
             + jnp.dot(y1_ref[...], w1[...], preferred_element_type=jnp.float32)
             + jnp.dot(y2_ref[...], w2[...], preferred_element_type=jnp.float32)
             + b_ref[...])
        z = y - jnp.max(y)
        e = jnp.exp(z)
        out_ref[...] = e / jnp.sum(e)

    return pl.pallas_call(
        body, out_shape=jax.ShapeDtypeStruct((1, OUTD), jnp.float32))(
        y0, y1, y2, Wc[0:10], Wc[10:20], Wc[20:30], bc[None, :])


def _pad_edge_arr(a, e_pad, dtype):
    E = a.shape[0]
    a = jnp.pad(a.astype(dtype), (0, e_pad - E))
    return a.reshape(e_pad // 128, 128)


def _pad_w(w, fout=FP):
    return jnp.pad(w, ((0, 0), (0, fout - w.shape[1])))


def _pad_w2(w):
    return jnp.pad(w, ((0, FP - w.shape[0]), (0, FP - w.shape[1])))


def _pad_v(v):
    return jnp.pad(v, (0, FP - v.shape[0]))


def _san_weights(p, first):
    ld = p["l_d"]
    W = _pad_w(ld["W"]) if first else _pad_w2(ld["W"])
    pW = _pad_w(p["p_W"]) if first else _pad_w2(p["p_W"])
    return (W, _pad_v(ld["b"])[None, :], pW, _pad_v(p["p_b"])[None, :],
            _pad_v(ld["a_src"])[None, :], _pad_v(ld["a_dst"])[None, :])


def _run_level(X, params, lvl, gat_idx_list, p_idx, p_val, n):
    n_pad = _n_pad(n)
    ep_p = _e_pad(p_idx.shape[1])
    ppack = jnp.stack([_pad_edge_arr(p_idx[1], ep_p, jnp.int32),
                       _pad_edge_arr(p_idx[0], ep_p, jnp.int32)], axis=1)
    pval = _pad_edge_arr(p_val, ep_p, jnp.float32)
    gats = []
    for gi in gat_idx_list:
        ep = _e_pad(gi.shape[1])
        gpack = jnp.stack([_pad_edge_arr(gi[0], ep, jnp.int32),
                           _pad_edge_arr(gi[1], ep, jnp.int32)], axis=1)
        gats.append((gpack, ep, gi.shape[1]))

    dpart = _build_dseg(ep_p, n_pad)(ppack, pval)

    xs = []
    pw = None
    h = hp = a_s = a_d = cvec = None
    for k in (1, 2, 3):
        p = params["l%d_%d" % (lvl, k)]
        if k == 1:
            W, b, pW, pb, asrc, adst = _san_weights(p, True)
            h, hp = _build_affine_first(n, X.shape[1])(X, W, b, pW, pb)
            a_s, a_d, cvec = _build_attn(n)(h, asrc, adst)
            dinv = _build_dinv(n, n_pad)(dpart)
            pw = _build_pvals(ep_p, n)(ppack, pval, dinv.reshape(n))
        parts = [_build_rows(ep_p, n, n_pad, False)(hp, ppack, pw)]
        for (gpack, ep, e_real) in gats:
            ex, den = _build_softmax(ep, e_real, n, n_pad)(
                gpack, a_s.reshape(n), a_d.reshape(n), cvec.reshape(n))
            parts.append(_build_rows(ep, n, n_pad, True)(h, gpack, ex, den))
        if k < 3:
            p2 = params["l%d_%d" % (lvl, k + 1)]
            W, b, pW, pb, asrc, adst = _san_weights(p2, False)
            x, h, hp = _build_affine_next(n, n_pad, len(parts))(
                *parts, W, b, pW, pb)
            a_s, a_d, cvec = _build_attn(n)(h, asrc, adst)
            xs.append(x)
        else:
            lin = params["l%d_4" % lvl]
            W4 = lin["W"]
            Wa = jnp.pad(W4[0:30], ((0, 2), (0, 0)))
            Wb = jnp.pad(W4[30:60], ((0, 2), (0, 0)))
            Wc = jnp.pad(W4[60:90], ((0, 2), (0, 0)))
            return _build_head(n, n_pad, len(parts))(
                xs[0], xs[1], *parts, Wa, Wb, Wc, lin["b"][None, :])


def kernel(X0, X1, X2, L0_idx, L0_val, L1u_idx, L1u_val, L1d_idx, L1d_val,
           L2_idx, L2_val, batch0, batch1, batch2, params):
    L1_idx = jnp.concatenate([L1u_idx, L1d_idx], axis=1)
    L1_val = jnp.concatenate([L1u_val, L1d_val], axis=0)
    y0 = _run_level(X0, params, 0, [L0_idx], L0_idx, L0_val, X0.shape[0])
    y1 = _run_level(X1, params, 1, [L1u_idx, L1d_idx], L1_idx, L1_val, X1.shape[0])
    y2 = _run_level(X2, params, 2, [L2_idx], L2_idx, L2_val, X2.shape[0])
    comb = params["combined"]
    return _final_combine(y0, y1, y2, comb["W"], comb["b"])
